# Initial kernel scaffold; baseline (speedup 1.0000x reference)
import functools

import jax
import jax.numpy as jnp
from jax.experimental import pallas as pl
from jax.experimental.pallas import tpu as pltpu

T = 2048; D = 2048; H = 16; HD = 128; E = 8; TOPK = 2; F = 1408; FS = 5632
BASE = 1000000.0; EPS = 1e-6

BT = 256
BQ = 512
BN = 512


def _rms(x, scale):
    return x * jax.lax.rsqrt(jnp.mean(x * x, axis=-1, keepdims=True) + EPS) * scale


def _qkv_body(hs_ref, w_ref, b_ref, ln_ref, out_ref):
    h = _rms(hs_ref[:], ln_ref[:])
    out_ref[:] = jnp.dot(h, w_ref[:], preferred_element_type=jnp.float32) + b_ref[:]


def _qkv(hidden_states, wqkv, bqkv, ln1):
    grid = (T // BT, (3 * H * HD) // BN)
    return pl.pallas_call(
        _qkv_body,
        grid=grid,
        in_specs=[
            pl.BlockSpec((BT, D), lambda i, j: (i, 0)),
            pl.BlockSpec((D, BN), lambda i, j: (0, j)),
            pl.BlockSpec((1, BN), lambda i, j: (0, j)),
            pl.BlockSpec((1, D), lambda i, j: (0, 0)),
        ],
        out_specs=pl.BlockSpec((BT, BN), lambda i, j: (i, j)),
        out_shape=jax.ShapeDtypeStruct((T, 3 * H * HD), jnp.float32),
        compiler_params=pltpu.CompilerParams(
            dimension_semantics=("parallel", "parallel")),
    )(hidden_states, wqkv, bqkv, ln1)


def _rope_apply(x, cos, sin):
    x1 = x[:, :HD // 2]
    x2 = x[:, HD // 2:]
    return jnp.concatenate([x1 * cos - x2 * sin, x2 * cos + x1 * sin], axis=-1)


def _attn_body(q_ref, k_ref, v_ref, cosq_ref, sinq_ref, cos_ref, sin_ref,
               out_ref, s_ref):
    qb = pl.program_id(1)
    scale = 1.0 / (HD ** 0.5)
    q = _rope_apply(q_ref[:], cosq_ref[:], sinq_ref[:]) * scale
    k = _rope_apply(k_ref[:], cos_ref[:], sin_ref[:])

    row = qb * BQ + jax.lax.broadcasted_iota(jnp.int32, (BQ, BQ), 0)

    def fill(j, _):
        kj = jax.lax.dynamic_slice(k, (j * BQ, 0), (BQ, HD))
        s = jax.lax.dot_general(q, kj, (((1,), (1,)), ((), ())),
                                preferred_element_type=jnp.float32)
        col = j * BQ + jax.lax.broadcasted_iota(jnp.int32, (BQ, BQ), 1)
        s = jnp.where(row >= col, s, -1e30)
        s_ref[:, pl.ds(j * BQ, BQ)] = s
        return 0

    s_ref[:] = jnp.full((BQ, T), -1e30, jnp.float32)
    jax.lax.fori_loop(0, qb + 1, fill, 0)

    s = s_ref[:]
    m = jnp.max(s, axis=-1, keepdims=True)
    p = jnp.exp(s - m)
    p = p / jnp.sum(p, axis=-1, keepdims=True)
    s_ref[:] = p

    def accum(j, acc):
        pj = s_ref[:, pl.ds(j * BQ, BQ)]
        vj = jax.lax.dynamic_slice(v_ref[:], (j * BQ, 0), (BQ, HD))
        return acc + jnp.dot(pj, vj, preferred_element_type=jnp.float32)

    out_ref[:] = jax.lax.fori_loop(0, qb + 1, accum,
                                   jnp.zeros((BQ, HD), jnp.float32))


def _attention(qkv, cos, sin):
    grid = (H, T // BQ)
    return pl.pallas_call(
        _attn_body,
        grid=grid,
        in_specs=[
            pl.BlockSpec((BQ, HD), lambda h, qb: (qb, h)),
            pl.BlockSpec((T, HD), lambda h, qb: (0, H + h)),
            pl.BlockSpec((T, HD), lambda h, qb: (0, 2 * H + h)),
            pl.BlockSpec((BQ, HD // 2), lambda h, qb: (qb, 0)),
            pl.BlockSpec((BQ, HD // 2), lambda h, qb: (qb, 0)),
            pl.BlockSpec((T, HD // 2), lambda h, qb: (0, 0)),
            pl.BlockSpec((T, HD // 2), lambda h, qb: (0, 0)),
        ],
        out_specs=pl.BlockSpec((BQ, HD), lambda h, qb: (qb, h)),
        out_shape=jax.ShapeDtypeStruct((T, H * HD), jnp.float32),
        scratch_shapes=[pltpu.VMEM((BQ, T), jnp.float32)],
        compiler_params=pltpu.CompilerParams(
            dimension_semantics=("parallel", "arbitrary")),
    )(qkv, qkv, qkv, cos, sin, cos, sin)


def _wo_body(a_ref, w_ref, r_ref, out_ref):
    out_ref[:] = (jnp.dot(a_ref[:], w_ref[:], preferred_element_type=jnp.float32)
                  + r_ref[:])


def _wo_proj(attn, wo, residual):
    grid = (T // BT, D // BN)
    return pl.pallas_call(
        _wo_body,
        grid=grid,
        in_specs=[
            pl.BlockSpec((BT, H * HD), lambda i, j: (i, 0)),
            pl.BlockSpec((H * HD, BN), lambda i, j: (0, j)),
            pl.BlockSpec((BT, BN), lambda i, j: (i, j)),
        ],
        out_specs=pl.BlockSpec((BT, BN), lambda i, j: (i, j)),
        out_shape=jax.ShapeDtypeStruct((T, D), jnp.float32),
        compiler_params=pltpu.CompilerParams(
            dimension_semantics=("parallel", "parallel")),
    )(attn, wo, residual)


def _rms2_body(h_ref, ln_ref, sgw_ref, h2_ref, gate_ref):
    h2 = _rms(h_ref[:], ln_ref[:])
    h2_ref[:] = h2
    gate_ref[:] = jax.nn.sigmoid(
        jnp.dot(h2, sgw_ref[:], preferred_element_type=jnp.float32))


def _rms2(h, ln2, sgw):
    grid = (T // BT,)
    return pl.pallas_call(
        _rms2_body,
        grid=grid,
        in_specs=[
            pl.BlockSpec((BT, D), lambda i: (i, 0)),
            pl.BlockSpec((1, D), lambda i: (0, 0)),
            pl.BlockSpec((D, 1), lambda i: (0, 0)),
        ],
        out_specs=[
            pl.BlockSpec((BT, D), lambda i: (i, 0)),
            pl.BlockSpec((BT, 1), lambda i: (i, 0)),
        ],
        out_shape=[
            jax.ShapeDtypeStruct((T, D), jnp.float32),
            jax.ShapeDtypeStruct((T, 1), jnp.float32),
        ],
        compiler_params=pltpu.CompilerParams(
            dimension_semantics=("parallel",)),
    )(h, ln2, sgw)


def _sh1_body(h2_ref, sg_ref, su_ref, act_ref):
    g = jnp.dot(h2_ref[:], sg_ref[:], preferred_element_type=jnp.float32)
    u = jnp.dot(h2_ref[:], su_ref[:], preferred_element_type=jnp.float32)
    act_ref[:] = g * jax.nn.sigmoid(g) * u


def _shared1(h2, sg, su):
    grid = (FS // BN,)
    return pl.pallas_call(
        _sh1_body,
        grid=grid,
        in_specs=[
            pl.BlockSpec((T, D), lambda j: (0, 0)),
            pl.BlockSpec((D, BN), lambda j: (0, j)),
            pl.BlockSpec((D, BN), lambda j: (0, j)),
        ],
        out_specs=pl.BlockSpec((T, BN), lambda j: (0, j)),
        out_shape=jax.ShapeDtypeStruct((T, FS), jnp.float32),
        compiler_params=pltpu.CompilerParams(
            dimension_semantics=("arbitrary",)),
    )(h2, sg, su)


def _sh2_body(a_ref, sd_ref, gate_ref, out_ref):
    out_ref[:] = gate_ref[:] * jnp.dot(a_ref[:], sd_ref[:],
                                       preferred_element_type=jnp.float32)


def _shared2(act, sd, gate):
    grid = (D // BN, T // BT)
    return pl.pallas_call(
        _sh2_body,
        grid=grid,
        in_specs=[
            pl.BlockSpec((BT, FS), lambda j, i: (i, 0)),
            pl.BlockSpec((FS, BN), lambda j, i: (0, j)),
            pl.BlockSpec((BT, 1), lambda j, i: (i, 0)),
        ],
        out_specs=pl.BlockSpec((BT, BN), lambda j, i: (i, j)),
        out_shape=jax.ShapeDtypeStruct((T, D), jnp.float32),
        compiler_params=pltpu.CompilerParams(
            dimension_semantics=("arbitrary", "arbitrary")),
    )(act, sd, gate)


def _router_body(h2_ref, rw_ref, cw_ref):
    logits = jnp.dot(h2_ref[:], rw_ref[:], preferred_element_type=jnp.float32)
    p = jax.nn.softmax(logits, axis=-1)
    iota = jax.lax.broadcasted_iota(jnp.int32, p.shape, 1)
    m1 = jnp.max(p, axis=-1, keepdims=True)
    i1 = jnp.min(jnp.where(p == m1, iota, E), axis=-1, keepdims=True)
    p2 = jnp.where(iota == i1, -1.0, p)
    m2 = jnp.max(p2, axis=-1, keepdims=True)
    i2 = jnp.min(jnp.where(p2 == m2, iota, E), axis=-1, keepdims=True)
    denom = m1 + m2
    cw_ref[:] = jnp.where(iota == i1, m1 / denom,
                          jnp.where(iota == i2, m2 / denom, 0.0))


def _router(h2, router_w):
    grid = (T // BT,)
    return pl.pallas_call(
        _router_body,
        grid=grid,
        in_specs=[
            pl.BlockSpec((BT, D), lambda i: (i, 0)),
            pl.BlockSpec((D, E), lambda i: (0, 0)),
        ],
        out_specs=pl.BlockSpec((BT, E), lambda i: (i, 0)),
        out_shape=jax.ShapeDtypeStruct((T, E), jnp.float32),
        compiler_params=pltpu.CompilerParams(
            dimension_semantics=("parallel",)),
    )(h2, router_w)


def _moe1_body(h2_ref, eg_ref, eu_ref, cw_ref, act_ref):
    g = jnp.dot(h2_ref[:], eg_ref[0], preferred_element_type=jnp.float32)
    u = jnp.dot(h2_ref[:], eu_ref[0], preferred_element_type=jnp.float32)
    act_ref[0] = (g * jax.nn.sigmoid(g) * u) * cw_ref[:]


def _moe1(h2, eg, eu, cw):
    grid = (E, T // BT)
    return pl.pallas_call(
        _moe1_body,
        grid=grid,
        in_specs=[
            pl.BlockSpec((BT, D), lambda e, i: (i, 0)),
            pl.BlockSpec((1, D, F), lambda e, i: (e, 0, 0)),
            pl.BlockSpec((1, D, F), lambda e, i: (e, 0, 0)),
            pl.BlockSpec((BT, 1), lambda e, i: (i, e)),
        ],
        out_specs=pl.BlockSpec((1, BT, F), lambda e, i: (e, i, 0)),
        out_shape=jax.ShapeDtypeStruct((E, T, F), jnp.float32),
        compiler_params=pltpu.CompilerParams(
            dimension_semantics=("arbitrary", "arbitrary")),
    )(h2, eg, eu, cw)


def _moe2_body(act_ref, ed_ref, sh_ref, out_ref):
    k = pl.program_id(1)

    @pl.when(k == 0)
    def _():
        out_ref[:] = sh_ref[:]

    out_ref[:] += jnp.dot(act_ref[0], ed_ref[0],
                          preferred_element_type=jnp.float32)


def _moe2(act, ed, shared):
    BI = 1024
    grid = (T // BI, E)
    return pl.pallas_call(
        _moe2_body,
        grid=grid,
        in_specs=[
            pl.BlockSpec((1, BI, F), lambda i, k: (k, i, 0)),
            pl.BlockSpec((1, F, D), lambda i, k: (k, 0, 0)),
            pl.BlockSpec((BI, D), lambda i, k: (i, 0)),
        ],
        out_specs=pl.BlockSpec((BI, D), lambda i, k: (i, 0)),
        out_shape=jax.ShapeDtypeStruct((T, D), jnp.float32),
        compiler_params=pltpu.CompilerParams(
            dimension_semantics=("arbitrary", "arbitrary")),
    )(act, ed, shared)


def kernel(positions, hidden_states, wq, bq, wk, bk, wv, bv, wo, ln1, ln2,
           router_w, eg, eu, ed, sg, su, sd, sgw):
    half = HD // 2
    inv = 1.0 / (BASE ** (jnp.arange(half, dtype=jnp.float32) / half))
    ang = positions.astype(jnp.float32)[:, None] * inv[None, :]
    cos = jnp.cos(ang)
    sin = jnp.sin(ang)

    wqkv = jnp.concatenate([wq, wk, wv], axis=1)
    bqkv = jnp.concatenate([bq, bk, bv]).reshape(1, -1)
    ln1r = ln1.reshape(1, D)
    ln2r = ln2.reshape(1, D)

    qkv = _qkv(hidden_states, wqkv, bqkv, ln1r)
    attn = _attention(qkv, cos, sin)
    h1 = _wo_proj(attn, wo, hidden_states)
    h2, gate = _rms2(h1, ln2r, sgw)
    act_s = _shared1(h2, sg, su)
    shared = _shared2(act_s, sd, gate)
    cw = _router(h2, router_w)
    act_e = _moe1(h2, eg, eu, cw)
    out = _moe2(act_e, ed, shared)
    return (out, h1)


# speedup vs baseline: 1.0033x; 1.0033x over previous
import functools

import jax
import jax.numpy as jnp
from jax.experimental import pallas as pl
from jax.experimental.pallas import tpu as pltpu

T = 2048; D = 2048; H = 16; HD = 128; E = 8; TOPK = 2; F = 1408; FS = 5632
BASE = 1000000.0; EPS = 1e-6

BT = 256
BQ = 512
BN = 512


def _rms(x, scale):
    return x * jax.lax.rsqrt(jnp.mean(x * x, axis=-1, keepdims=True) + EPS) * scale


def _qkv_body(hs_ref, w_ref, b_ref, ln_ref, out_ref):
    h = _rms(hs_ref[:], ln_ref[:])
    out_ref[:] = jnp.dot(h, w_ref[:], preferred_element_type=jnp.float32) + b_ref[:]


def _qkv(hidden_states, wqkv, bqkv, ln1):
    grid = (T // BT, (3 * H * HD) // BN)
    return pl.pallas_call(
        _qkv_body,
        grid=grid,
        in_specs=[
            pl.BlockSpec((BT, D), lambda i, j: (i, 0)),
            pl.BlockSpec((D, BN), lambda i, j: (0, j)),
            pl.BlockSpec((1, BN), lambda i, j: (0, j)),
            pl.BlockSpec((1, D), lambda i, j: (0, 0)),
        ],
        out_specs=pl.BlockSpec((BT, BN), lambda i, j: (i, j)),
        out_shape=jax.ShapeDtypeStruct((T, 3 * H * HD), jnp.float32),
        compiler_params=pltpu.CompilerParams(
            dimension_semantics=("parallel", "parallel")),
    )(hidden_states, wqkv, bqkv, ln1)


def _rope_apply(x, cos, sin):
    x1 = x[:, :HD // 2]
    x2 = x[:, HD // 2:]
    return jnp.concatenate([x1 * cos - x2 * sin, x2 * cos + x1 * sin], axis=-1)


def _attn_body(q_ref, k_ref, v_ref, cosq_ref, sinq_ref, cos_ref, sin_ref,
               out_ref, s_ref, kr_ref):
    qb = pl.program_id(1)
    scale = 1.0 / (HD ** 0.5)
    q = _rope_apply(q_ref[:], cosq_ref[:], sinq_ref[:])
    kr_ref[:] = _rope_apply(k_ref[:], cos_ref[:], sin_ref[:])

    row = qb * BQ + jax.lax.broadcasted_iota(jnp.int32, (BQ, BQ), 0)

    def fill(j, _):
        kj = kr_ref[pl.ds(j * BQ, BQ), :]
        s = jax.lax.dot_general(q, kj, (((1,), (1,)), ((), ())),
                                preferred_element_type=jnp.float32) * scale
        col = j * BQ + jax.lax.broadcasted_iota(jnp.int32, (BQ, BQ), 1)
        s = jnp.where(row >= col, s, -1e30)
        s_ref[:, pl.ds(j * BQ, BQ)] = s
        return 0

    s_ref[:] = jnp.full((BQ, T), -1e30, jnp.float32)
    jax.lax.fori_loop(0, qb + 1, fill, 0)

    s = s_ref[:]
    m = jnp.max(s, axis=-1, keepdims=True)
    p = jnp.exp(s - m)
    p = p / jnp.sum(p, axis=-1, keepdims=True)
    s_ref[:] = p

    def accum(j, acc):
        pj = s_ref[:, pl.ds(j * BQ, BQ)]
        vj = v_ref[pl.ds(j * BQ, BQ), :]
        return acc + jnp.dot(pj, vj, preferred_element_type=jnp.float32)

    out_ref[:] = jax.lax.fori_loop(0, qb + 1, accum,
                                   jnp.zeros((BQ, HD), jnp.float32))


def _attention(qkv, cos, sin):
    grid = (H, T // BQ)
    return pl.pallas_call(
        _attn_body,
        grid=grid,
        in_specs=[
            pl.BlockSpec((BQ, HD), lambda h, qb: (qb, h)),
            pl.BlockSpec((T, HD), lambda h, qb: (0, H + h)),
            pl.BlockSpec((T, HD), lambda h, qb: (0, 2 * H + h)),
            pl.BlockSpec((BQ, HD // 2), lambda h, qb: (qb, 0)),
            pl.BlockSpec((BQ, HD // 2), lambda h, qb: (qb, 0)),
            pl.BlockSpec((T, HD // 2), lambda h, qb: (0, 0)),
            pl.BlockSpec((T, HD // 2), lambda h, qb: (0, 0)),
        ],
        out_specs=pl.BlockSpec((BQ, HD), lambda h, qb: (qb, h)),
        out_shape=jax.ShapeDtypeStruct((T, H * HD), jnp.float32),
        scratch_shapes=[pltpu.VMEM((BQ, T), jnp.float32),
                        pltpu.VMEM((T, HD), jnp.float32)],
        compiler_params=pltpu.CompilerParams(
            dimension_semantics=("parallel", "arbitrary")),
    )(qkv, qkv, qkv, cos, sin, cos, sin)


def _wo_body(a_ref, w_ref, r_ref, out_ref):
    out_ref[:] = (jnp.dot(a_ref[:], w_ref[:], preferred_element_type=jnp.float32)
                  + r_ref[:])


def _wo_proj(attn, wo, residual):
    grid = (T // BT, D // BN)
    return pl.pallas_call(
        _wo_body,
        grid=grid,
        in_specs=[
            pl.BlockSpec((BT, H * HD), lambda i, j: (i, 0)),
            pl.BlockSpec((H * HD, BN), lambda i, j: (0, j)),
            pl.BlockSpec((BT, BN), lambda i, j: (i, j)),
        ],
        out_specs=pl.BlockSpec((BT, BN), lambda i, j: (i, j)),
        out_shape=jax.ShapeDtypeStruct((T, D), jnp.float32),
        compiler_params=pltpu.CompilerParams(
            dimension_semantics=("parallel", "parallel")),
    )(attn, wo, residual)


def _rms2_body(h_ref, ln_ref, sgw_ref, h2_ref, gate_ref):
    h2 = _rms(h_ref[:], ln_ref[:])
    h2_ref[:] = h2
    gate_ref[:] = jax.nn.sigmoid(
        jnp.dot(h2, sgw_ref[:], preferred_element_type=jnp.float32))


def _rms2(h, ln2, sgw):
    grid = (T // BT,)
    return pl.pallas_call(
        _rms2_body,
        grid=grid,
        in_specs=[
            pl.BlockSpec((BT, D), lambda i: (i, 0)),
            pl.BlockSpec((1, D), lambda i: (0, 0)),
            pl.BlockSpec((D, 1), lambda i: (0, 0)),
        ],
        out_specs=[
            pl.BlockSpec((BT, D), lambda i: (i, 0)),
            pl.BlockSpec((BT, 1), lambda i: (i, 0)),
        ],
        out_shape=[
            jax.ShapeDtypeStruct((T, D), jnp.float32),
            jax.ShapeDtypeStruct((T, 1), jnp.float32),
        ],
        compiler_params=pltpu.CompilerParams(
            dimension_semantics=("parallel",)),
    )(h, ln2, sgw)


def _sh1_body(h2_ref, sg_ref, su_ref, act_ref):
    g = jnp.dot(h2_ref[:], sg_ref[:], preferred_element_type=jnp.float32)
    u = jnp.dot(h2_ref[:], su_ref[:], preferred_element_type=jnp.float32)
    act_ref[:] = g * jax.nn.sigmoid(g) * u


def _shared1(h2, sg, su):
    grid = (FS // BN, T // BQ)
    return pl.pallas_call(
        _sh1_body,
        grid=grid,
        in_specs=[
            pl.BlockSpec((BQ, D), lambda j, i: (i, 0)),
            pl.BlockSpec((D, BN), lambda j, i: (0, j)),
            pl.BlockSpec((D, BN), lambda j, i: (0, j)),
        ],
        out_specs=pl.BlockSpec((BQ, BN), lambda j, i: (i, j)),
        out_shape=jax.ShapeDtypeStruct((T, FS), jnp.float32),
        compiler_params=pltpu.CompilerParams(
            dimension_semantics=("arbitrary", "arbitrary")),
    )(h2, sg, su)


def _sh2_body(a_ref, sd_ref, gate_ref, out_ref):
    out_ref[:] = gate_ref[:] * jnp.dot(a_ref[:], sd_ref[:],
                                       preferred_element_type=jnp.float32)


def _shared2(act, sd, gate):
    grid = (D // BN, T // BT)
    return pl.pallas_call(
        _sh2_body,
        grid=grid,
        in_specs=[
            pl.BlockSpec((BT, FS), lambda j, i: (i, 0)),
            pl.BlockSpec((FS, BN), lambda j, i: (0, j)),
            pl.BlockSpec((BT, 1), lambda j, i: (i, 0)),
        ],
        out_specs=pl.BlockSpec((BT, BN), lambda j, i: (i, j)),
        out_shape=jax.ShapeDtypeStruct((T, D), jnp.float32),
        compiler_params=pltpu.CompilerParams(
            dimension_semantics=("arbitrary", "arbitrary")),
    )(act, sd, gate)


def _router_body(h2_ref, rw_ref, cw_ref):
    logits = jnp.dot(h2_ref[:], rw_ref[:], preferred_element_type=jnp.float32)
    p = jax.nn.softmax(logits, axis=-1)
    iota = jax.lax.broadcasted_iota(jnp.int32, p.shape, 1)
    m1 = jnp.max(p, axis=-1, keepdims=True)
    i1 = jnp.min(jnp.where(p == m1, iota, E), axis=-1, keepdims=True)
    p2 = jnp.where(iota == i1, -1.0, p)
    m2 = jnp.max(p2, axis=-1, keepdims=True)
    i2 = jnp.min(jnp.where(p2 == m2, iota, E), axis=-1, keepdims=True)
    denom = m1 + m2
    cw_ref[:] = jnp.where(iota == i1, m1 / denom,
                          jnp.where(iota == i2, m2 / denom, 0.0))


def _router(h2, router_w):
    grid = (T // BT,)
    return pl.pallas_call(
        _router_body,
        grid=grid,
        in_specs=[
            pl.BlockSpec((BT, D), lambda i: (i, 0)),
            pl.BlockSpec((D, E), lambda i: (0, 0)),
        ],
        out_specs=pl.BlockSpec((BT, E), lambda i: (i, 0)),
        out_shape=jax.ShapeDtypeStruct((T, E), jnp.float32),
        compiler_params=pltpu.CompilerParams(
            dimension_semantics=("parallel",)),
    )(h2, router_w)


def _moe1_body(h2_ref, eg_ref, eu_ref, cw_ref, act_ref, g_ref, u_ref):
    e = pl.program_id(0)
    k2 = pl.program_id(2)

    @pl.when(k2 == 0)
    def _():
        g_ref[:] = jnp.zeros_like(g_ref)
        u_ref[:] = jnp.zeros_like(u_ref)

    g_ref[:] += jnp.dot(h2_ref[:], eg_ref[0], preferred_element_type=jnp.float32)
    u_ref[:] += jnp.dot(h2_ref[:], eu_ref[0], preferred_element_type=jnp.float32)

    @pl.when(k2 == pl.num_programs(2) - 1)
    def _():
        g = g_ref[:]
        u = u_ref[:]
        cw = cw_ref[:]
        lane = jax.lax.broadcasted_iota(jnp.int32, cw.shape, 1)
        col = jnp.sum(jnp.where(lane == e, cw, 0.0), axis=-1, keepdims=True)
        act_ref[0] = (g * jax.nn.sigmoid(g) * u) * col


def _moe1(h2, eg, eu, cw):
    BD = 1024
    grid = (E, T // BT, D // BD)
    return pl.pallas_call(
        _moe1_body,
        grid=grid,
        in_specs=[
            pl.BlockSpec((BT, BD), lambda e, i, k2: (i, k2)),
            pl.BlockSpec((1, BD, F), lambda e, i, k2: (e, k2, 0)),
            pl.BlockSpec((1, BD, F), lambda e, i, k2: (e, k2, 0)),
            pl.BlockSpec((BT, E), lambda e, i, k2: (i, 0)),
        ],
        out_specs=pl.BlockSpec((1, BT, F), lambda e, i, k2: (e, i, 0)),
        out_shape=jax.ShapeDtypeStruct((E, T, F), jnp.float32),
        scratch_shapes=[pltpu.VMEM((BT, F), jnp.float32),
                        pltpu.VMEM((BT, F), jnp.float32)],
        compiler_params=pltpu.CompilerParams(
            dimension_semantics=("arbitrary", "arbitrary", "arbitrary")),
    )(h2, eg, eu, cw)


def _moe2_body(act_ref, ed_ref, sh_ref, out_ref):
    k = pl.program_id(1)

    @pl.when(k == 0)
    def _():
        out_ref[:] = sh_ref[:]

    out_ref[:] += jnp.dot(act_ref[0], ed_ref[0],
                          preferred_element_type=jnp.float32)


def _moe2(act, ed, shared):
    BI = 512
    grid = (T // BI, E)
    return pl.pallas_call(
        _moe2_body,
        grid=grid,
        in_specs=[
            pl.BlockSpec((1, BI, F), lambda i, k: (k, i, 0)),
            pl.BlockSpec((1, F, D), lambda i, k: (k, 0, 0)),
            pl.BlockSpec((BI, D), lambda i, k: (i, 0)),
        ],
        out_specs=pl.BlockSpec((BI, D), lambda i, k: (i, 0)),
        out_shape=jax.ShapeDtypeStruct((T, D), jnp.float32),
        compiler_params=pltpu.CompilerParams(
            dimension_semantics=("arbitrary", "arbitrary")),
    )(act, ed, shared)


def kernel(positions, hidden_states, wq, bq, wk, bk, wv, bv, wo, ln1, ln2,
           router_w, eg, eu, ed, sg, su, sd, sgw):
    half = HD // 2
    inv = 1.0 / (BASE ** (jnp.arange(half, dtype=jnp.float32) / half))
    ang = positions.astype(jnp.float32)[:, None] * inv[None, :]
    cos = jnp.cos(ang)
    sin = jnp.sin(ang)

    wqkv = jnp.concatenate([wq, wk, wv], axis=1)
    bqkv = jnp.concatenate([bq, bk, bv]).reshape(1, -1)
    ln1r = ln1.reshape(1, D)
    ln2r = ln2.reshape(1, D)

    qkv = _qkv(hidden_states, wqkv, bqkv, ln1r)
    attn = _attention(qkv, cos, sin)
    h1 = _wo_proj(attn, wo, hidden_states)
    h2, gate = _rms2(h1, ln2r, sgw)
    act_s = _shared1(h2, sg, su)
    shared = _shared2(act_s, sd, gate)
    cw = _router(h2, router_w)
    act_e = _moe1(h2, eg, eu, cw)
    out = _moe2(act_e, ed, shared)
    return (out, h1)


# speedup vs baseline: 1.1290x; 1.1252x over previous
import functools

import jax
import jax.numpy as jnp
from jax.experimental import pallas as pl
from jax.experimental.pallas import tpu as pltpu

T = 2048; D = 2048; H = 16; HD = 128; E = 8; TOPK = 2; F = 1408; FS = 5632
BASE = 1000000.0; EPS = 1e-6

BT = 256
BQ = 512
BN = 512


def _rms(x, scale):
    return x * jax.lax.rsqrt(jnp.mean(x * x, axis=-1, keepdims=True) + EPS) * scale


def _qkv_body(hs_ref, w_ref, b_ref, ln_ref, out_ref):
    h = _rms(hs_ref[:], ln_ref[:])
    out_ref[:] = jnp.dot(h, w_ref[:], preferred_element_type=jnp.float32) + b_ref[:]


def _qkv(hidden_states, wqkv, bqkv, ln1):
    grid = (T // BT, (3 * H * HD) // BN)
    return pl.pallas_call(
        _qkv_body,
        grid=grid,
        in_specs=[
            pl.BlockSpec((BT, D), lambda i, j: (i, 0)),
            pl.BlockSpec((D, BN), lambda i, j: (0, j)),
            pl.BlockSpec((1, BN), lambda i, j: (0, j)),
            pl.BlockSpec((1, D), lambda i, j: (0, 0)),
        ],
        out_specs=pl.BlockSpec((BT, BN), lambda i, j: (i, j)),
        out_shape=jax.ShapeDtypeStruct((T, 3 * H * HD), jnp.float32),
        compiler_params=pltpu.CompilerParams(
            dimension_semantics=("parallel", "parallel")),
    )(hidden_states, wqkv, bqkv, ln1)


def _rope_apply(x, cos, sin):
    x1 = x[:, :HD // 2]
    x2 = x[:, HD // 2:]
    return jnp.concatenate([x1 * cos - x2 * sin, x2 * cos + x1 * sin], axis=-1)


def _attn_body(q_ref, k_ref, v_ref, cosq_ref, sinq_ref, cos_ref, sin_ref,
               out_ref, s_ref, kr_ref):
    qb = pl.program_id(1)
    scale = 1.0 / (HD ** 0.5)
    q = _rope_apply(q_ref[:], cosq_ref[:], sinq_ref[:])
    kr_ref[:] = _rope_apply(k_ref[:], cos_ref[:], sin_ref[:])

    row = qb * BQ + jax.lax.broadcasted_iota(jnp.int32, (BQ, BQ), 0)

    def fill(j, _):
        kj = kr_ref[pl.ds(j * BQ, BQ), :]
        s = jax.lax.dot_general(q, kj, (((1,), (1,)), ((), ())),
                                preferred_element_type=jnp.float32) * scale
        col = j * BQ + jax.lax.broadcasted_iota(jnp.int32, (BQ, BQ), 1)
        s = jnp.where(row >= col, s, -1e30)
        s_ref[:, pl.ds(j * BQ, BQ)] = s
        return 0

    s_ref[:] = jnp.full((BQ, T), -1e30, jnp.float32)
    jax.lax.fori_loop(0, qb + 1, fill, 0)

    s = s_ref[:]
    m = jnp.max(s, axis=-1, keepdims=True)
    p = jnp.exp(s - m)
    p = p / jnp.sum(p, axis=-1, keepdims=True)
    s_ref[:] = p

    def accum(j, acc):
        pj = s_ref[:, pl.ds(j * BQ, BQ)]
        vj = v_ref[pl.ds(j * BQ, BQ), :]
        return acc + jnp.dot(pj, vj, preferred_element_type=jnp.float32)

    out_ref[:] = jax.lax.fori_loop(0, qb + 1, accum,
                                   jnp.zeros((BQ, HD), jnp.float32))


def _attention(qkv, cos, sin):
    grid = (H, T // BQ)
    return pl.pallas_call(
        _attn_body,
        grid=grid,
        in_specs=[
            pl.BlockSpec((BQ, HD), lambda h, qb: (qb, h)),
            pl.BlockSpec((T, HD), lambda h, qb: (0, H + h)),
            pl.BlockSpec((T, HD), lambda h, qb: (0, 2 * H + h)),
            pl.BlockSpec((BQ, HD // 2), lambda h, qb: (qb, 0)),
            pl.BlockSpec((BQ, HD // 2), lambda h, qb: (qb, 0)),
            pl.BlockSpec((T, HD // 2), lambda h, qb: (0, 0)),
            pl.BlockSpec((T, HD // 2), lambda h, qb: (0, 0)),
        ],
        out_specs=pl.BlockSpec((BQ, HD), lambda h, qb: (qb, h)),
        out_shape=jax.ShapeDtypeStruct((T, H * HD), jnp.float32),
        scratch_shapes=[pltpu.VMEM((BQ, T), jnp.float32),
                        pltpu.VMEM((T, HD), jnp.float32)],
        compiler_params=pltpu.CompilerParams(
            dimension_semantics=("parallel", "arbitrary")),
    )(qkv, qkv, qkv, cos, sin, cos, sin)


def _wo_body(a_ref, w_ref, r_ref, out_ref):
    out_ref[:] = (jnp.dot(a_ref[:], w_ref[:], preferred_element_type=jnp.float32)
                  + r_ref[:])


def _wo_proj(attn, wo, residual):
    grid = (T // BT, D // BN)
    return pl.pallas_call(
        _wo_body,
        grid=grid,
        in_specs=[
            pl.BlockSpec((BT, H * HD), lambda i, j: (i, 0)),
            pl.BlockSpec((H * HD, BN), lambda i, j: (0, j)),
            pl.BlockSpec((BT, BN), lambda i, j: (i, j)),
        ],
        out_specs=pl.BlockSpec((BT, BN), lambda i, j: (i, j)),
        out_shape=jax.ShapeDtypeStruct((T, D), jnp.float32),
        compiler_params=pltpu.CompilerParams(
            dimension_semantics=("parallel", "parallel")),
    )(attn, wo, residual)


def _rms2_body(h_ref, ln_ref, sgw_ref, h2_ref, gate_ref):
    h2 = _rms(h_ref[:], ln_ref[:])
    h2_ref[:] = h2
    gate_ref[:] = jax.nn.sigmoid(
        jnp.dot(h2, sgw_ref[:], preferred_element_type=jnp.float32))


def _rms2(h, ln2, sgw):
    grid = (T // BT,)
    return pl.pallas_call(
        _rms2_body,
        grid=grid,
        in_specs=[
            pl.BlockSpec((BT, D), lambda i: (i, 0)),
            pl.BlockSpec((1, D), lambda i: (0, 0)),
            pl.BlockSpec((D, 1), lambda i: (0, 0)),
        ],
        out_specs=[
            pl.BlockSpec((BT, D), lambda i: (i, 0)),
            pl.BlockSpec((BT, 1), lambda i: (i, 0)),
        ],
        out_shape=[
            jax.ShapeDtypeStruct((T, D), jnp.float32),
            jax.ShapeDtypeStruct((T, 1), jnp.float32),
        ],
        compiler_params=pltpu.CompilerParams(
            dimension_semantics=("parallel",)),
    )(h, ln2, sgw)


def _sh1_body(h2_ref, sg_ref, su_ref, act_ref):
    g = jnp.dot(h2_ref[:], sg_ref[:], preferred_element_type=jnp.float32)
    u = jnp.dot(h2_ref[:], su_ref[:], preferred_element_type=jnp.float32)
    act_ref[:] = g * jax.nn.sigmoid(g) * u


def _shared1(h2, sg, su):
    grid = (FS // BN, T // BQ)
    return pl.pallas_call(
        _sh1_body,
        grid=grid,
        in_specs=[
            pl.BlockSpec((BQ, D), lambda j, i: (i, 0)),
            pl.BlockSpec((D, BN), lambda j, i: (0, j)),
            pl.BlockSpec((D, BN), lambda j, i: (0, j)),
        ],
        out_specs=pl.BlockSpec((BQ, BN), lambda j, i: (i, j)),
        out_shape=jax.ShapeDtypeStruct((T, FS), jnp.float32),
        compiler_params=pltpu.CompilerParams(
            dimension_semantics=("arbitrary", "arbitrary")),
    )(h2, sg, su)


def _sh2_body(a_ref, sd_ref, gate_ref, out_ref):
    out_ref[:] = gate_ref[:] * jnp.dot(a_ref[:], sd_ref[:],
                                       preferred_element_type=jnp.float32)


def _shared2(act, sd, gate):
    grid = (D // BN, T // BT)
    return pl.pallas_call(
        _sh2_body,
        grid=grid,
        in_specs=[
            pl.BlockSpec((BT, FS), lambda j, i: (i, 0)),
            pl.BlockSpec((FS, BN), lambda j, i: (0, j)),
            pl.BlockSpec((BT, 1), lambda j, i: (i, 0)),
        ],
        out_specs=pl.BlockSpec((BT, BN), lambda j, i: (i, j)),
        out_shape=jax.ShapeDtypeStruct((T, D), jnp.float32),
        compiler_params=pltpu.CompilerParams(
            dimension_semantics=("arbitrary", "arbitrary")),
    )(act, sd, gate)


BTM = 128
NT = 2 * T // BTM + E
NPAD = NT * BTM
NW = 32


def _cumsum_rows(x):
    n = x.shape[0]
    sh = 1
    while sh < n:
        x = x + jnp.concatenate(
            [jnp.zeros((sh, x.shape[1]), x.dtype), x[:-sh]], axis=0)
        sh *= 2
    return x


def _router_body(h2_ref, rw_ref, pos0_ref, pos1_ref, w0_ref, w1_ref, et_ref):
    logits = jnp.dot(h2_ref[:], rw_ref[:], preferred_element_type=jnp.float32)
    p = jax.nn.softmax(logits, axis=-1)
    iota = jax.lax.broadcasted_iota(jnp.int32, p.shape, 1)
    m1 = jnp.max(p, axis=-1, keepdims=True)
    i1 = jnp.min(jnp.where(p == m1, iota, E), axis=-1, keepdims=True)
    p2 = jnp.where(iota == i1, -1.0, p)
    m2 = jnp.max(p2, axis=-1, keepdims=True)
    i2 = jnp.min(jnp.where(p2 == m2, iota, E), axis=-1, keepdims=True)
    denom = m1 + m2
    w0_ref[:] = m1 / denom
    w1_ref[:] = m2 / denom

    oh0 = (iota == i1).astype(jnp.float32)
    oh1 = (iota == i2).astype(jnp.float32)
    inc0 = _cumsum_rows(oh0)
    inc1 = _cumsum_rows(oh1)
    cnt0 = inc0[T - 1:, :]
    counts = cnt0 + inc1[T - 1:, :]
    counts_i = counts.astype(jnp.int32)
    tiles = ((counts_i + (BTM - 1)) // BTM).astype(jnp.float32)

    r = jax.lax.broadcasted_iota(jnp.int32, (E, E), 0)
    c = jax.lax.broadcasted_iota(jnp.int32, (E, E), 1)
    ustrict = (r < c).astype(jnp.float32)
    uincl = (r <= c).astype(jnp.float32)
    start_tile = jnp.dot(tiles, ustrict, preferred_element_type=jnp.float32)
    end_tile = jnp.dot(tiles, uincl, preferred_element_type=jnp.float32)
    pad_start = start_tile * float(BTM)

    rank0 = inc0 - oh0
    rank1 = cnt0 + inc1 - oh1
    pos0 = jnp.sum(oh0 * (pad_start + rank0), axis=1, keepdims=True)
    pos1 = jnp.sum(oh1 * (pad_start + rank1), axis=1, keepdims=True)
    pos0_ref[:] = pos0.astype(jnp.int32)
    pos1_ref[:] = pos1.astype(jnp.int32)

    g = jax.lax.broadcasted_iota(jnp.int32, (NT, E), 0)
    ind = (g >= end_tile.astype(jnp.int32)).astype(jnp.float32)
    et = jnp.sum(ind, axis=1, keepdims=True).astype(jnp.int32)
    et_ref[:] = jnp.minimum(et, E - 1)


def _router(h2, router_w):
    return pl.pallas_call(
        _router_body,
        grid=(1,),
        in_specs=[
            pl.BlockSpec((T, D), lambda i: (0, 0)),
            pl.BlockSpec((D, E), lambda i: (0, 0)),
        ],
        out_specs=[
            pl.BlockSpec((T, 1), lambda i: (0, 0)),
            pl.BlockSpec((T, 1), lambda i: (0, 0)),
            pl.BlockSpec((T, 1), lambda i: (0, 0)),
            pl.BlockSpec((T, 1), lambda i: (0, 0)),
            pl.BlockSpec((NT, 1), lambda i: (0, 0)),
        ],
        out_shape=[
            jax.ShapeDtypeStruct((T, 1), jnp.int32),
            jax.ShapeDtypeStruct((T, 1), jnp.int32),
            jax.ShapeDtypeStruct((T, 1), jnp.float32),
            jax.ShapeDtypeStruct((T, 1), jnp.float32),
            jax.ShapeDtypeStruct((NT, 1), jnp.int32),
        ],
        compiler_params=pltpu.CompilerParams(
            dimension_semantics=("arbitrary",)),
    )(h2, router_w)


def _sc_dispatch(h2, w16, pos3d):
    from jax.experimental.pallas import tpu_sc as plsc

    mesh = plsc.VectorSubcoreMesh(core_axis_name="c", subcore_axis_name="s")

    @functools.partial(
        pl.kernel, mesh=mesh,
        out_type=[jax.ShapeDtypeStruct((NPAD, D), jnp.float32),
                  jax.ShapeDtypeStruct((NPAD, 128), jnp.float32)],
        scratch_types=[pltpu.VMEM((8, 16), jnp.int32),
                       pltpu.VMEM((128, 128), jnp.float32),
                       pltpu.VMEM((16, D), jnp.float32),
                       pltpu.SemaphoreType.DMA,
                       pltpu.SemaphoreType.DMA],
    )
    def disp(h2_hbm, w16_hbm, pos_hbm, x_hbm, ws_hbm, posv, wbuf, rows,
             sem1, sem2):
        w = jax.lax.axis_index("s") * 2 + jax.lax.axis_index("c")
        base = w * 128
        pltpu.sync_copy(pos_hbm.at[w], posv)
        pltpu.sync_copy(w16_hbm.at[pl.ds(base, 128)], wbuf)
        for j in range(8):
            tok = jax.lax.rem(base + j * 16, T)
            pltpu.sync_copy(h2_hbm.at[pl.ds(tok, 16)], rows)
            pltpu.async_copy(rows, x_hbm.at[posv.at[j]], sem1).wait()
            pltpu.async_copy(wbuf.at[pl.ds(j * 16, 16)],
                             ws_hbm.at[posv.at[j]], sem2).wait()

    return disp(h2, w16, pos3d)


def _gmoe1_body(et_ref, x_ref, eg_ref, eu_ref, act_ref, g_ref, u_ref):
    k2 = pl.program_id(1)

    @pl.when(k2 == 0)
    def _():
        g_ref[:] = jnp.zeros_like(g_ref)
        u_ref[:] = jnp.zeros_like(u_ref)

    g_ref[:] += jnp.dot(x_ref[:], eg_ref[0], preferred_element_type=jnp.float32)
    u_ref[:] += jnp.dot(x_ref[:], eu_ref[0], preferred_element_type=jnp.float32)

    @pl.when(k2 == pl.num_programs(1) - 1)
    def _():
        g = g_ref[:]
        act_ref[:] = g * jax.nn.sigmoid(g) * u_ref[:]


def _gmoe1(etile, x, eg, eu):
    BD = 1024
    grid_spec = pltpu.PrefetchScalarGridSpec(
        num_scalar_prefetch=1,
        grid=(NT, D // BD),
        in_specs=[
            pl.BlockSpec((BTM, BD), lambda g, k2, et: (g, k2)),
            pl.BlockSpec((1, BD, F), lambda g, k2, et: (et[g], k2, 0)),
            pl.BlockSpec((1, BD, F), lambda g, k2, et: (et[g], k2, 0)),
        ],
        out_specs=pl.BlockSpec((BTM, F), lambda g, k2, et: (g, 0)),
        scratch_shapes=[pltpu.VMEM((BTM, F), jnp.float32),
                        pltpu.VMEM((BTM, F), jnp.float32)],
    )
    return pl.pallas_call(
        _gmoe1_body,
        grid_spec=grid_spec,
        out_shape=jax.ShapeDtypeStruct((NPAD, F), jnp.float32),
        compiler_params=pltpu.CompilerParams(
            dimension_semantics=("arbitrary", "arbitrary")),
    )(etile, x, eg, eu)


def _gmoe2_body(et_ref, a_ref, ed_ref, ws_ref, y_ref):
    y = jnp.dot(a_ref[:], ed_ref[0], preferred_element_type=jnp.float32)
    y_ref[:] = y * ws_ref[:, :1]


def _gmoe2(etile, act, ed, ws):
    grid_spec = pltpu.PrefetchScalarGridSpec(
        num_scalar_prefetch=1,
        grid=(NT,),
        in_specs=[
            pl.BlockSpec((BTM, F), lambda g, et: (g, 0)),
            pl.BlockSpec((1, F, D), lambda g, et: (et[g], 0, 0)),
            pl.BlockSpec((BTM, 128), lambda g, et: (g, 0)),
        ],
        out_specs=pl.BlockSpec((BTM, D), lambda g, et: (g, 0)),
    )
    return pl.pallas_call(
        _gmoe2_body,
        grid_spec=grid_spec,
        out_shape=jax.ShapeDtypeStruct((NPAD, D), jnp.float32),
        compiler_params=pltpu.CompilerParams(
            dimension_semantics=("arbitrary",)),
    )(etile, act, ed, ws)


def _sc_combine(y, shared, p03d, p13d):
    from jax.experimental.pallas import tpu_sc as plsc

    mesh = plsc.VectorSubcoreMesh(core_axis_name="c", subcore_axis_name="s")

    @functools.partial(
        pl.kernel, mesh=mesh,
        out_type=jax.ShapeDtypeStruct((T, D), jnp.float32),
        scratch_types=[pltpu.VMEM((4, 16), jnp.int32),
                       pltpu.VMEM((4, 16), jnp.int32),
                       pltpu.VMEM((16, D), jnp.float32),
                       pltpu.VMEM((16, D), jnp.float32),
                       pltpu.VMEM((16, D), jnp.float32),
                       pltpu.SemaphoreType.DMA,
                       pltpu.SemaphoreType.DMA],
    )
    def comb(y_hbm, sh_hbm, p0_hbm, p1_hbm, out_hbm, p0v, p1v, y0, y1, acc,
             sem1, sem2):
        w = jax.lax.axis_index("s") * 2 + jax.lax.axis_index("c")
        pltpu.sync_copy(p0_hbm.at[w], p0v)
        pltpu.sync_copy(p1_hbm.at[w], p1v)
        for j in range(4):
            tok = w * 64 + j * 16
            cp0 = pltpu.async_copy(y_hbm.at[p0v.at[j]], y0, sem1)
            cp1 = pltpu.async_copy(y_hbm.at[p1v.at[j]], y1, sem2)
            pltpu.sync_copy(sh_hbm.at[pl.ds(tok, 16)], acc)
            cp0.wait()
            cp1.wait()
            for i in range(16):
                def body(cc, carry):
                    sl = pl.ds(cc * 16, 16)
                    acc[i, sl] = acc[i, sl] + y0[i, sl] + y1[i, sl]
                    return carry
                jax.lax.fori_loop(0, D // 16, body, 0)
            pltpu.sync_copy(acc, out_hbm.at[pl.ds(tok, 16)])

    return comb(y, shared, p03d, p13d)


def kernel(positions, hidden_states, wq, bq, wk, bk, wv, bv, wo, ln1, ln2,
           router_w, eg, eu, ed, sg, su, sd, sgw):
    half = HD // 2
    inv = 1.0 / (BASE ** (jnp.arange(half, dtype=jnp.float32) / half))
    ang = positions.astype(jnp.float32)[:, None] * inv[None, :]
    cos = jnp.cos(ang)
    sin = jnp.sin(ang)

    wqkv = jnp.concatenate([wq, wk, wv], axis=1)
    bqkv = jnp.concatenate([bq, bk, bv]).reshape(1, -1)
    ln1r = ln1.reshape(1, D)
    ln2r = ln2.reshape(1, D)

    qkv = _qkv(hidden_states, wqkv, bqkv, ln1r)
    attn = _attention(qkv, cos, sin)
    h1 = _wo_proj(attn, wo, hidden_states)
    h2, gate = _rms2(h1, ln2r, sgw)
    act_s = _shared1(h2, sg, su)
    shared = _shared2(act_s, sd, gate)

    pos0, pos1, w0, w1, etile = _router(h2, router_w)

    pos3d = jnp.concatenate([pos0, pos1], axis=0).reshape(NW, 8, 16)
    w16 = jnp.tile(jnp.concatenate([w0, w1], axis=0), (1, 128))
    p03d = pos0.reshape(NW, 4, 16)
    p13d = pos1.reshape(NW, 4, 16)

    x, ws = _sc_dispatch(h2, w16, pos3d)
    act_e = _gmoe1(etile.reshape(NT), x, eg, eu)
    y = _gmoe2(etile.reshape(NT), act_e, ed, ws)
    out = _sc_combine(y, shared, p03d, p13d)
    return (out, h1)


# speedup vs baseline: 1.2759x; 1.1301x over previous
import functools

import jax
import jax.numpy as jnp
from jax.experimental import pallas as pl
from jax.experimental.pallas import tpu as pltpu

T = 2048; D = 2048; H = 16; HD = 128; E = 8; TOPK = 2; F = 1408; FS = 5632
BASE = 1000000.0; EPS = 1e-6

BT = 256
BQ = 512
BN = 512


def _rms(x, scale):
    return x * jax.lax.rsqrt(jnp.mean(x * x, axis=-1, keepdims=True) + EPS) * scale


def _qkv_body(hs_ref, w_ref, b_ref, ln_ref, out_ref):
    h = _rms(hs_ref[:], ln_ref[:])
    out_ref[:] = jnp.dot(h, w_ref[:], preferred_element_type=jnp.float32) + b_ref[:]


def _qkv(hidden_states, wqkv, bqkv, ln1):
    grid = (T // BT, (3 * H * HD) // BN)
    return pl.pallas_call(
        _qkv_body,
        grid=grid,
        in_specs=[
            pl.BlockSpec((BT, D), lambda i, j: (i, 0)),
            pl.BlockSpec((D, BN), lambda i, j: (0, j)),
            pl.BlockSpec((1, BN), lambda i, j: (0, j)),
            pl.BlockSpec((1, D), lambda i, j: (0, 0)),
        ],
        out_specs=pl.BlockSpec((BT, BN), lambda i, j: (i, j)),
        out_shape=jax.ShapeDtypeStruct((T, 3 * H * HD), jnp.float32),
        compiler_params=pltpu.CompilerParams(
            dimension_semantics=("parallel", "parallel")),
    )(hidden_states, wqkv, bqkv, ln1)


def _rope_apply(x, cos, sin):
    x1 = x[:, :HD // 2]
    x2 = x[:, HD // 2:]
    return jnp.concatenate([x1 * cos - x2 * sin, x2 * cos + x1 * sin], axis=-1)


def _attn_body(q_ref, k_ref, v_ref, cosq_ref, sinq_ref, cos_ref, sin_ref,
               out_ref, s_ref, kr_ref):
    qb = pl.program_id(1)
    scale = 1.0 / (HD ** 0.5)
    q = _rope_apply(q_ref[:], cosq_ref[:], sinq_ref[:])
    kr_ref[:] = _rope_apply(k_ref[:], cos_ref[:], sin_ref[:])

    row = qb * BQ + jax.lax.broadcasted_iota(jnp.int32, (BQ, BQ), 0)

    def fill(j, _):
        kj = kr_ref[pl.ds(j * BQ, BQ), :]
        s = jax.lax.dot_general(q, kj, (((1,), (1,)), ((), ())),
                                preferred_element_type=jnp.float32) * scale
        col = j * BQ + jax.lax.broadcasted_iota(jnp.int32, (BQ, BQ), 1)
        s = jnp.where(row >= col, s, -1e30)
        s_ref[:, pl.ds(j * BQ, BQ)] = s
        return 0

    s_ref[:] = jnp.full((BQ, T), -1e30, jnp.float32)
    jax.lax.fori_loop(0, qb + 1, fill, 0)

    s = s_ref[:]
    m = jnp.max(s, axis=-1, keepdims=True)
    p = jnp.exp(s - m)
    p = p / jnp.sum(p, axis=-1, keepdims=True)
    s_ref[:] = p

    def accum(j, acc):
        pj = s_ref[:, pl.ds(j * BQ, BQ)]
        vj = v_ref[pl.ds(j * BQ, BQ), :]
        return acc + jnp.dot(pj, vj, preferred_element_type=jnp.float32)

    out_ref[:] = jax.lax.fori_loop(0, qb + 1, accum,
                                   jnp.zeros((BQ, HD), jnp.float32))


def _attention(qkv, cos, sin):
    grid = (H, T // BQ)
    return pl.pallas_call(
        _attn_body,
        grid=grid,
        in_specs=[
            pl.BlockSpec((BQ, HD), lambda h, qb: (qb, h)),
            pl.BlockSpec((T, HD), lambda h, qb: (0, H + h)),
            pl.BlockSpec((T, HD), lambda h, qb: (0, 2 * H + h)),
            pl.BlockSpec((BQ, HD // 2), lambda h, qb: (qb, 0)),
            pl.BlockSpec((BQ, HD // 2), lambda h, qb: (qb, 0)),
            pl.BlockSpec((T, HD // 2), lambda h, qb: (0, 0)),
            pl.BlockSpec((T, HD // 2), lambda h, qb: (0, 0)),
        ],
        out_specs=pl.BlockSpec((BQ, HD), lambda h, qb: (qb, h)),
        out_shape=jax.ShapeDtypeStruct((T, H * HD), jnp.float32),
        scratch_shapes=[pltpu.VMEM((BQ, T), jnp.float32),
                        pltpu.VMEM((T, HD), jnp.float32)],
        compiler_params=pltpu.CompilerParams(
            dimension_semantics=("parallel", "arbitrary")),
    )(qkv, qkv, qkv, cos, sin, cos, sin)


def _wo_body(a_ref, w_ref, r_ref, out_ref):
    out_ref[:] = (jnp.dot(a_ref[:], w_ref[:], preferred_element_type=jnp.float32)
                  + r_ref[:])


def _wo_proj(attn, wo, residual):
    grid = (T // BT, D // BN)
    return pl.pallas_call(
        _wo_body,
        grid=grid,
        in_specs=[
            pl.BlockSpec((BT, H * HD), lambda i, j: (i, 0)),
            pl.BlockSpec((H * HD, BN), lambda i, j: (0, j)),
            pl.BlockSpec((BT, BN), lambda i, j: (i, j)),
        ],
        out_specs=pl.BlockSpec((BT, BN), lambda i, j: (i, j)),
        out_shape=jax.ShapeDtypeStruct((T, D), jnp.float32),
        compiler_params=pltpu.CompilerParams(
            dimension_semantics=("parallel", "parallel")),
    )(attn, wo, residual)


def _rms2_body(h_ref, ln_ref, sgw_ref, h2_ref, gate_ref):
    h2 = _rms(h_ref[:], ln_ref[:])
    h2_ref[:] = h2
    gate_ref[:] = jax.nn.sigmoid(
        jnp.dot(h2, sgw_ref[:], preferred_element_type=jnp.float32))


def _rms2(h, ln2, sgw):
    grid = (T // BT,)
    return pl.pallas_call(
        _rms2_body,
        grid=grid,
        in_specs=[
            pl.BlockSpec((BT, D), lambda i: (i, 0)),
            pl.BlockSpec((1, D), lambda i: (0, 0)),
            pl.BlockSpec((D, 1), lambda i: (0, 0)),
        ],
        out_specs=[
            pl.BlockSpec((BT, D), lambda i: (i, 0)),
            pl.BlockSpec((BT, 1), lambda i: (i, 0)),
        ],
        out_shape=[
            jax.ShapeDtypeStruct((T, D), jnp.float32),
            jax.ShapeDtypeStruct((T, 1), jnp.float32),
        ],
        compiler_params=pltpu.CompilerParams(
            dimension_semantics=("parallel",)),
    )(h, ln2, sgw)


def _sh1_body(h2_ref, sg_ref, su_ref, act_ref):
    g = jnp.dot(h2_ref[:], sg_ref[:], preferred_element_type=jnp.float32)
    u = jnp.dot(h2_ref[:], su_ref[:], preferred_element_type=jnp.float32)
    act_ref[:] = g * jax.nn.sigmoid(g) * u


def _shared1(h2, sg, su):
    grid = (FS // BN, T // BQ)
    return pl.pallas_call(
        _sh1_body,
        grid=grid,
        in_specs=[
            pl.BlockSpec((BQ, D), lambda j, i: (i, 0)),
            pl.BlockSpec((D, BN), lambda j, i: (0, j)),
            pl.BlockSpec((D, BN), lambda j, i: (0, j)),
        ],
        out_specs=pl.BlockSpec((BQ, BN), lambda j, i: (i, j)),
        out_shape=jax.ShapeDtypeStruct((T, FS), jnp.float32),
        compiler_params=pltpu.CompilerParams(
            dimension_semantics=("arbitrary", "arbitrary")),
    )(h2, sg, su)


def _sh2_body(a_ref, sd_ref, gate_ref, out_ref):
    out_ref[:] = gate_ref[:] * jnp.dot(a_ref[:], sd_ref[:],
                                       preferred_element_type=jnp.float32)


def _shared2(act, sd, gate):
    grid = (D // BN, T // BT)
    return pl.pallas_call(
        _sh2_body,
        grid=grid,
        in_specs=[
            pl.BlockSpec((BT, FS), lambda j, i: (i, 0)),
            pl.BlockSpec((FS, BN), lambda j, i: (0, j)),
            pl.BlockSpec((BT, 1), lambda j, i: (i, 0)),
        ],
        out_specs=pl.BlockSpec((BT, BN), lambda j, i: (i, j)),
        out_shape=jax.ShapeDtypeStruct((T, D), jnp.float32),
        compiler_params=pltpu.CompilerParams(
            dimension_semantics=("arbitrary", "arbitrary")),
    )(act, sd, gate)


BTM = 128
NT = 2 * T // BTM + E
NPAD = NT * BTM
NW = 32


def _cumsum_rows(x):
    n = x.shape[0]
    sh = 1
    while sh < n:
        x = x + jnp.concatenate(
            [jnp.zeros((sh, x.shape[1]), x.dtype), x[:-sh]], axis=0)
        sh *= 2
    return x


def _router_body(h2_ref, rw_ref, pos0_ref, pos1_ref, w0_ref, w1_ref, et_ref):
    logits = jnp.dot(h2_ref[:], rw_ref[:], preferred_element_type=jnp.float32)
    p = jax.nn.softmax(logits, axis=-1)
    iota = jax.lax.broadcasted_iota(jnp.int32, p.shape, 1)
    m1 = jnp.max(p, axis=-1, keepdims=True)
    i1 = jnp.min(jnp.where(p == m1, iota, E), axis=-1, keepdims=True)
    p2 = jnp.where(iota == i1, -1.0, p)
    m2 = jnp.max(p2, axis=-1, keepdims=True)
    i2 = jnp.min(jnp.where(p2 == m2, iota, E), axis=-1, keepdims=True)
    denom = m1 + m2
    w0_ref[:] = m1 / denom
    w1_ref[:] = m2 / denom

    oh0 = (iota == i1).astype(jnp.float32)
    oh1 = (iota == i2).astype(jnp.float32)
    inc0 = _cumsum_rows(oh0)
    inc1 = _cumsum_rows(oh1)
    cnt0 = inc0[T - 1:, :]
    counts = cnt0 + inc1[T - 1:, :]
    counts_i = counts.astype(jnp.int32)
    tiles = ((counts_i + (BTM - 1)) // BTM).astype(jnp.float32)

    r = jax.lax.broadcasted_iota(jnp.int32, (E, E), 0)
    c = jax.lax.broadcasted_iota(jnp.int32, (E, E), 1)
    ustrict = (r < c).astype(jnp.float32)
    uincl = (r <= c).astype(jnp.float32)
    start_tile = jnp.dot(tiles, ustrict, preferred_element_type=jnp.float32)
    end_tile = jnp.dot(tiles, uincl, preferred_element_type=jnp.float32)
    pad_start = start_tile * float(BTM)

    rank0 = inc0 - oh0
    rank1 = cnt0 + inc1 - oh1
    pos0 = jnp.sum(oh0 * (pad_start + rank0), axis=1, keepdims=True)
    pos1 = jnp.sum(oh1 * (pad_start + rank1), axis=1, keepdims=True)
    pos0_ref[:] = pos0.astype(jnp.int32)
    pos1_ref[:] = pos1.astype(jnp.int32)

    g = jax.lax.broadcasted_iota(jnp.int32, (NT, E), 0)
    ind = (g >= end_tile.astype(jnp.int32)).astype(jnp.float32)
    et = jnp.sum(ind, axis=1, keepdims=True).astype(jnp.int32)
    et_ref[:] = jnp.minimum(et, E - 1)


def _router(h2, router_w):
    return pl.pallas_call(
        _router_body,
        grid=(1,),
        in_specs=[
            pl.BlockSpec((T, D), lambda i: (0, 0)),
            pl.BlockSpec((D, E), lambda i: (0, 0)),
        ],
        out_specs=[
            pl.BlockSpec((T, 1), lambda i: (0, 0)),
            pl.BlockSpec((T, 1), lambda i: (0, 0)),
            pl.BlockSpec((T, 1), lambda i: (0, 0)),
            pl.BlockSpec((T, 1), lambda i: (0, 0)),
            pl.BlockSpec((NT, 1), lambda i: (0, 0)),
        ],
        out_shape=[
            jax.ShapeDtypeStruct((T, 1), jnp.int32),
            jax.ShapeDtypeStruct((T, 1), jnp.int32),
            jax.ShapeDtypeStruct((T, 1), jnp.float32),
            jax.ShapeDtypeStruct((T, 1), jnp.float32),
            jax.ShapeDtypeStruct((NT, 1), jnp.int32),
        ],
        compiler_params=pltpu.CompilerParams(
            dimension_semantics=("arbitrary",)),
    )(h2, router_w)


def _sc_dispatch(h2, w16, pos3d):
    from jax.experimental.pallas import tpu_sc as plsc

    mesh = plsc.VectorSubcoreMesh(core_axis_name="c", subcore_axis_name="s")

    @functools.partial(
        pl.kernel, mesh=mesh,
        out_type=[jax.ShapeDtypeStruct((NPAD, D), jnp.float32),
                  jax.ShapeDtypeStruct((NPAD, 128), jnp.float32)],
        scratch_types=[pltpu.VMEM((8, 16), jnp.int32),
                       pltpu.VMEM((128, 128), jnp.float32),
                       pltpu.VMEM((16, D), jnp.float32),
                       pltpu.SemaphoreType.DMA,
                       pltpu.SemaphoreType.DMA],
    )
    def disp(h2_hbm, w16_hbm, pos_hbm, x_hbm, ws_hbm, posv, wbuf, rows,
             sem1, sem2):
        w = jax.lax.axis_index("s") * 2 + jax.lax.axis_index("c")
        base = w * 128
        pltpu.sync_copy(pos_hbm.at[w], posv)
        pltpu.sync_copy(w16_hbm.at[pl.ds(base, 128)], wbuf)
        for j in range(8):
            tok = jax.lax.rem(base + j * 16, T)
            pltpu.sync_copy(h2_hbm.at[pl.ds(tok, 16)], rows)
            pltpu.async_copy(rows, x_hbm.at[posv.at[j]], sem1).wait()
            pltpu.async_copy(wbuf.at[pl.ds(j * 16, 16)],
                             ws_hbm.at[posv.at[j]], sem2).wait()

    return disp(h2, w16, pos3d)


def _gmoe1_body(et_ref, x_ref, eg_ref, eu_ref, act_ref):
    g = jnp.dot(x_ref[:], eg_ref[0], preferred_element_type=jnp.float32)
    u = jnp.dot(x_ref[:], eu_ref[0], preferred_element_type=jnp.float32)
    act_ref[:] = g * jax.nn.sigmoid(g) * u


def _gmoe1(etile, x, eg, eu):
    grid_spec = pltpu.PrefetchScalarGridSpec(
        num_scalar_prefetch=1,
        grid=(NT,),
        in_specs=[
            pl.BlockSpec((BTM, D), lambda g, et: (g, 0)),
            pl.BlockSpec((1, D, F), lambda g, et: (et[g], 0, 0)),
            pl.BlockSpec((1, D, F), lambda g, et: (et[g], 0, 0)),
        ],
        out_specs=pl.BlockSpec((BTM, F), lambda g, et: (g, 0)),
    )
    return pl.pallas_call(
        _gmoe1_body,
        grid_spec=grid_spec,
        out_shape=jax.ShapeDtypeStruct((NPAD, F), jnp.float32),
        compiler_params=pltpu.CompilerParams(
            dimension_semantics=("arbitrary",)),
    )(etile, x, eg, eu)


def _gmoe2_body(et_ref, a_ref, ed_ref, ws_ref, y_ref):
    y = jnp.dot(a_ref[:], ed_ref[0], preferred_element_type=jnp.float32)
    y_ref[:] = y * ws_ref[:, :1]


def _gmoe2(etile, act, ed, ws):
    grid_spec = pltpu.PrefetchScalarGridSpec(
        num_scalar_prefetch=1,
        grid=(NT,),
        in_specs=[
            pl.BlockSpec((BTM, F), lambda g, et: (g, 0)),
            pl.BlockSpec((1, F, D), lambda g, et: (et[g], 0, 0)),
            pl.BlockSpec((BTM, 128), lambda g, et: (g, 0)),
        ],
        out_specs=pl.BlockSpec((BTM, D), lambda g, et: (g, 0)),
    )
    return pl.pallas_call(
        _gmoe2_body,
        grid_spec=grid_spec,
        out_shape=jax.ShapeDtypeStruct((NPAD, D), jnp.float32),
        compiler_params=pltpu.CompilerParams(
            dimension_semantics=("arbitrary",)),
    )(etile, act, ed, ws)


def _sc_combine(y, shared, p03d, p13d):
    from jax.experimental.pallas import tpu_sc as plsc

    mesh = plsc.VectorSubcoreMesh(core_axis_name="c", subcore_axis_name="s")

    @functools.partial(
        pl.kernel, mesh=mesh,
        out_type=jax.ShapeDtypeStruct((T, D), jnp.float32),
        scratch_types=[pltpu.VMEM((4, 16), jnp.int32),
                       pltpu.VMEM((4, 16), jnp.int32),
                       pltpu.VMEM((16, D), jnp.float32),
                       pltpu.VMEM((16, D), jnp.float32),
                       pltpu.VMEM((16, D), jnp.float32),
                       pltpu.SemaphoreType.DMA,
                       pltpu.SemaphoreType.DMA],
    )
    def comb(y_hbm, sh_hbm, p0_hbm, p1_hbm, out_hbm, p0v, p1v, y0, y1, acc,
             sem1, sem2):
        w = jax.lax.axis_index("s") * 2 + jax.lax.axis_index("c")
        pltpu.sync_copy(p0_hbm.at[w], p0v)
        pltpu.sync_copy(p1_hbm.at[w], p1v)
        for j in range(4):
            tok = w * 64 + j * 16
            cp0 = pltpu.async_copy(y_hbm.at[p0v.at[j]], y0, sem1)
            cp1 = pltpu.async_copy(y_hbm.at[p1v.at[j]], y1, sem2)
            pltpu.sync_copy(sh_hbm.at[pl.ds(tok, 16)], acc)
            cp0.wait()
            cp1.wait()
            for i in range(16):
                def body(cc, carry):
                    sl = pl.ds(cc * 16, 16)
                    acc[i, sl] = acc[i, sl] + y0[i, sl] + y1[i, sl]
                    return carry
                jax.lax.fori_loop(0, D // 16, body, 0)
            pltpu.sync_copy(acc, out_hbm.at[pl.ds(tok, 16)])

    return comb(y, shared, p03d, p13d)


def kernel(positions, hidden_states, wq, bq, wk, bk, wv, bv, wo, ln1, ln2,
           router_w, eg, eu, ed, sg, su, sd, sgw):
    half = HD // 2
    inv = 1.0 / (BASE ** (jnp.arange(half, dtype=jnp.float32) / half))
    ang = positions.astype(jnp.float32)[:, None] * inv[None, :]
    cos = jnp.cos(ang)
    sin = jnp.sin(ang)

    wqkv = jnp.concatenate([wq, wk, wv], axis=1)
    bqkv = jnp.concatenate([bq, bk, bv]).reshape(1, -1)
    ln1r = ln1.reshape(1, D)
    ln2r = ln2.reshape(1, D)

    qkv = _qkv(hidden_states, wqkv, bqkv, ln1r)
    attn = _attention(qkv, cos, sin)
    h1 = _wo_proj(attn, wo, hidden_states)
    h2, gate = _rms2(h1, ln2r, sgw)
    act_s = _shared1(h2, sg, su)
    shared = _shared2(act_s, sd, gate)

    pos0, pos1, w0, w1, etile = _router(h2, router_w)

    pos3d = jnp.concatenate([pos0, pos1], axis=0).reshape(NW, 8, 16)
    w16 = jnp.tile(jnp.concatenate([w0, w1], axis=0), (1, 128))
    p03d = pos0.reshape(NW, 4, 16)
    p13d = pos1.reshape(NW, 4, 16)

    x, ws = _sc_dispatch(h2, w16, pos3d)
    act_e = _gmoe1(etile.reshape(NT), x, eg, eu)
    y = _gmoe2(etile.reshape(NT), act_e, ed, ws)
    out = _sc_combine(y, shared, p03d, p13d)
    return (out, h1)


# speedup vs baseline: 1.2770x; 1.0009x over previous
import functools

import jax
import jax.numpy as jnp
from jax.experimental import pallas as pl
from jax.experimental.pallas import tpu as pltpu

T = 2048; D = 2048; H = 16; HD = 128; E = 8; TOPK = 2; F = 1408; FS = 5632
BASE = 1000000.0; EPS = 1e-6

BT = 256
BQ = 512
BN = 512


def _rms(x, scale):
    return x * jax.lax.rsqrt(jnp.mean(x * x, axis=-1, keepdims=True) + EPS) * scale


def _qkv_body(hs_ref, w_ref, b_ref, ln_ref, out_ref):
    h = _rms(hs_ref[:], ln_ref[:])
    out_ref[:] = jnp.dot(h, w_ref[:], preferred_element_type=jnp.float32) + b_ref[:]


def _qkv(hidden_states, wqkv, bqkv, ln1):
    grid = (T // BT, (3 * H * HD) // BN)
    return pl.pallas_call(
        _qkv_body,
        grid=grid,
        in_specs=[
            pl.BlockSpec((BT, D), lambda i, j: (i, 0)),
            pl.BlockSpec((D, BN), lambda i, j: (0, j)),
            pl.BlockSpec((1, BN), lambda i, j: (0, j)),
            pl.BlockSpec((1, D), lambda i, j: (0, 0)),
        ],
        out_specs=pl.BlockSpec((BT, BN), lambda i, j: (i, j)),
        out_shape=jax.ShapeDtypeStruct((T, 3 * H * HD), jnp.float32),
        compiler_params=pltpu.CompilerParams(
            dimension_semantics=("parallel", "parallel")),
    )(hidden_states, wqkv, bqkv, ln1)


def _rope_apply(x, cos, sin):
    x1 = x[:, :HD // 2]
    x2 = x[:, HD // 2:]
    return jnp.concatenate([x1 * cos - x2 * sin, x2 * cos + x1 * sin], axis=-1)


def _attn_body(q_ref, k_ref, v_ref, cosq_ref, sinq_ref, cos_ref, sin_ref,
               out_ref, s_ref, kr_ref):
    qb = pl.program_id(1)
    scale = 1.0 / (HD ** 0.5)
    q = _rope_apply(q_ref[:], cosq_ref[:], sinq_ref[:])
    kr_ref[:] = _rope_apply(k_ref[:], cos_ref[:], sin_ref[:])

    row = qb * BQ + jax.lax.broadcasted_iota(jnp.int32, (BQ, BQ), 0)

    def fill(j, _):
        kj = kr_ref[pl.ds(j * BQ, BQ), :]
        s = jax.lax.dot_general(q, kj, (((1,), (1,)), ((), ())),
                                preferred_element_type=jnp.float32) * scale
        col = j * BQ + jax.lax.broadcasted_iota(jnp.int32, (BQ, BQ), 1)
        s = jnp.where(row >= col, s, -1e30)
        s_ref[:, pl.ds(j * BQ, BQ)] = s
        return 0

    s_ref[:] = jnp.full((BQ, T), -1e30, jnp.float32)
    jax.lax.fori_loop(0, qb + 1, fill, 0)

    s = s_ref[:]
    m = jnp.max(s, axis=-1, keepdims=True)
    p = jnp.exp(s - m)
    p = p / jnp.sum(p, axis=-1, keepdims=True)
    s_ref[:] = p

    def accum(j, acc):
        pj = s_ref[:, pl.ds(j * BQ, BQ)]
        vj = v_ref[pl.ds(j * BQ, BQ), :]
        return acc + jnp.dot(pj, vj, preferred_element_type=jnp.float32)

    out_ref[:] = jax.lax.fori_loop(0, qb + 1, accum,
                                   jnp.zeros((BQ, HD), jnp.float32))


def _attention(qkv, cos, sin):
    grid = (H, T // BQ)
    return pl.pallas_call(
        _attn_body,
        grid=grid,
        in_specs=[
            pl.BlockSpec((BQ, HD), lambda h, qb: (qb, h)),
            pl.BlockSpec((T, HD), lambda h, qb: (0, H + h)),
            pl.BlockSpec((T, HD), lambda h, qb: (0, 2 * H + h)),
            pl.BlockSpec((BQ, HD // 2), lambda h, qb: (qb, 0)),
            pl.BlockSpec((BQ, HD // 2), lambda h, qb: (qb, 0)),
            pl.BlockSpec((T, HD // 2), lambda h, qb: (0, 0)),
            pl.BlockSpec((T, HD // 2), lambda h, qb: (0, 0)),
        ],
        out_specs=pl.BlockSpec((BQ, HD), lambda h, qb: (qb, h)),
        out_shape=jax.ShapeDtypeStruct((T, H * HD), jnp.float32),
        scratch_shapes=[pltpu.VMEM((BQ, T), jnp.float32),
                        pltpu.VMEM((T, HD), jnp.float32)],
        compiler_params=pltpu.CompilerParams(
            dimension_semantics=("parallel", "arbitrary")),
    )(qkv, qkv, qkv, cos, sin, cos, sin)


def _wo_body(a_ref, w_ref, r_ref, out_ref):
    out_ref[:] = (jnp.dot(a_ref[:], w_ref[:], preferred_element_type=jnp.float32)
                  + r_ref[:])


def _wo_proj(attn, wo, residual):
    grid = (T // BT, D // BN)
    return pl.pallas_call(
        _wo_body,
        grid=grid,
        in_specs=[
            pl.BlockSpec((BT, H * HD), lambda i, j: (i, 0)),
            pl.BlockSpec((H * HD, BN), lambda i, j: (0, j)),
            pl.BlockSpec((BT, BN), lambda i, j: (i, j)),
        ],
        out_specs=pl.BlockSpec((BT, BN), lambda i, j: (i, j)),
        out_shape=jax.ShapeDtypeStruct((T, D), jnp.float32),
        compiler_params=pltpu.CompilerParams(
            dimension_semantics=("parallel", "parallel")),
    )(attn, wo, residual)


def _rms2_body(h_ref, ln_ref, sgw_ref, h2_ref, gate_ref):
    h2 = _rms(h_ref[:], ln_ref[:])
    h2_ref[:] = h2
    gate_ref[:] = jax.nn.sigmoid(
        jnp.dot(h2, sgw_ref[:], preferred_element_type=jnp.float32))


def _rms2(h, ln2, sgw):
    grid = (T // BT,)
    return pl.pallas_call(
        _rms2_body,
        grid=grid,
        in_specs=[
            pl.BlockSpec((BT, D), lambda i: (i, 0)),
            pl.BlockSpec((1, D), lambda i: (0, 0)),
            pl.BlockSpec((D, 1), lambda i: (0, 0)),
        ],
        out_specs=[
            pl.BlockSpec((BT, D), lambda i: (i, 0)),
            pl.BlockSpec((BT, 1), lambda i: (i, 0)),
        ],
        out_shape=[
            jax.ShapeDtypeStruct((T, D), jnp.float32),
            jax.ShapeDtypeStruct((T, 1), jnp.float32),
        ],
        compiler_params=pltpu.CompilerParams(
            dimension_semantics=("parallel",)),
    )(h, ln2, sgw)


def _sh1_body(h2_ref, sg_ref, su_ref, act_ref):
    g = jnp.dot(h2_ref[:], sg_ref[:], preferred_element_type=jnp.float32)
    u = jnp.dot(h2_ref[:], su_ref[:], preferred_element_type=jnp.float32)
    act_ref[:] = g * jax.nn.sigmoid(g) * u


def _shared1(h2, sg, su):
    grid = (FS // BN, T // BQ)
    return pl.pallas_call(
        _sh1_body,
        grid=grid,
        in_specs=[
            pl.BlockSpec((BQ, D), lambda j, i: (i, 0)),
            pl.BlockSpec((D, BN), lambda j, i: (0, j)),
            pl.BlockSpec((D, BN), lambda j, i: (0, j)),
        ],
        out_specs=pl.BlockSpec((BQ, BN), lambda j, i: (i, j)),
        out_shape=jax.ShapeDtypeStruct((T, FS), jnp.float32),
        compiler_params=pltpu.CompilerParams(
            dimension_semantics=("arbitrary", "arbitrary")),
    )(h2, sg, su)


def _sh2_body(a_ref, sd_ref, gate_ref, out_ref):
    out_ref[:] = gate_ref[:] * jnp.dot(a_ref[:], sd_ref[:],
                                       preferred_element_type=jnp.float32)


def _shared2(act, sd, gate):
    grid = (D // BN, T // BT)
    return pl.pallas_call(
        _sh2_body,
        grid=grid,
        in_specs=[
            pl.BlockSpec((BT, FS), lambda j, i: (i, 0)),
            pl.BlockSpec((FS, BN), lambda j, i: (0, j)),
            pl.BlockSpec((BT, 1), lambda j, i: (i, 0)),
        ],
        out_specs=pl.BlockSpec((BT, BN), lambda j, i: (i, j)),
        out_shape=jax.ShapeDtypeStruct((T, D), jnp.float32),
        compiler_params=pltpu.CompilerParams(
            dimension_semantics=("arbitrary", "arbitrary")),
    )(act, sd, gate)


BTM = 256
NT = 2 * T // BTM + E
NPAD = NT * BTM
NW = 32


def _cumsum_rows(x):
    n = x.shape[0]
    sh = 1
    while sh < n:
        x = x + jnp.concatenate(
            [jnp.zeros((sh, x.shape[1]), x.dtype), x[:-sh]], axis=0)
        sh *= 2
    return x


def _router_body(h2_ref, rw_ref, pos0_ref, pos1_ref, w0_ref, w1_ref, et_ref):
    logits = jnp.dot(h2_ref[:], rw_ref[:], preferred_element_type=jnp.float32)
    p = jax.nn.softmax(logits, axis=-1)
    iota = jax.lax.broadcasted_iota(jnp.int32, p.shape, 1)
    m1 = jnp.max(p, axis=-1, keepdims=True)
    i1 = jnp.min(jnp.where(p == m1, iota, E), axis=-1, keepdims=True)
    p2 = jnp.where(iota == i1, -1.0, p)
    m2 = jnp.max(p2, axis=-1, keepdims=True)
    i2 = jnp.min(jnp.where(p2 == m2, iota, E), axis=-1, keepdims=True)
    denom = m1 + m2
    w0_ref[:] = m1 / denom
    w1_ref[:] = m2 / denom

    oh0 = (iota == i1).astype(jnp.float32)
    oh1 = (iota == i2).astype(jnp.float32)
    inc0 = _cumsum_rows(oh0)
    inc1 = _cumsum_rows(oh1)
    cnt0 = inc0[T - 1:, :]
    counts = cnt0 + inc1[T - 1:, :]
    counts_i = counts.astype(jnp.int32)
    tiles = ((counts_i + (BTM - 1)) // BTM).astype(jnp.float32)

    r = jax.lax.broadcasted_iota(jnp.int32, (E, E), 0)
    c = jax.lax.broadcasted_iota(jnp.int32, (E, E), 1)
    ustrict = (r < c).astype(jnp.float32)
    uincl = (r <= c).astype(jnp.float32)
    start_tile = jnp.dot(tiles, ustrict, preferred_element_type=jnp.float32)
    end_tile = jnp.dot(tiles, uincl, preferred_element_type=jnp.float32)
    pad_start = start_tile * float(BTM)

    rank0 = inc0 - oh0
    rank1 = cnt0 + inc1 - oh1
    pos0 = jnp.sum(oh0 * (pad_start + rank0), axis=1, keepdims=True)
    pos1 = jnp.sum(oh1 * (pad_start + rank1), axis=1, keepdims=True)
    pos0_ref[:] = pos0.astype(jnp.int32)
    pos1_ref[:] = pos1.astype(jnp.int32)

    g = jax.lax.broadcasted_iota(jnp.int32, (NT, E), 0)
    ind = (g >= end_tile.astype(jnp.int32)).astype(jnp.float32)
    et = jnp.sum(ind, axis=1, keepdims=True).astype(jnp.int32)
    et_ref[:] = jnp.minimum(et, E - 1)


def _router(h2, router_w):
    return pl.pallas_call(
        _router_body,
        grid=(1,),
        in_specs=[
            pl.BlockSpec((T, D), lambda i: (0, 0)),
            pl.BlockSpec((D, E), lambda i: (0, 0)),
        ],
        out_specs=[
            pl.BlockSpec((T, 1), lambda i: (0, 0)),
            pl.BlockSpec((T, 1), lambda i: (0, 0)),
            pl.BlockSpec((T, 1), lambda i: (0, 0)),
            pl.BlockSpec((T, 1), lambda i: (0, 0)),
            pl.BlockSpec((NT, 1), lambda i: (0, 0)),
        ],
        out_shape=[
            jax.ShapeDtypeStruct((T, 1), jnp.int32),
            jax.ShapeDtypeStruct((T, 1), jnp.int32),
            jax.ShapeDtypeStruct((T, 1), jnp.float32),
            jax.ShapeDtypeStruct((T, 1), jnp.float32),
            jax.ShapeDtypeStruct((NT, 1), jnp.int32),
        ],
        compiler_params=pltpu.CompilerParams(
            dimension_semantics=("arbitrary",)),
    )(h2, router_w)


def _sc_dispatch(h2, w16, pos3d):
    from jax.experimental.pallas import tpu_sc as plsc

    mesh = plsc.VectorSubcoreMesh(core_axis_name="c", subcore_axis_name="s")

    @functools.partial(
        pl.kernel, mesh=mesh,
        out_type=[jax.ShapeDtypeStruct((NPAD, D), jnp.float32),
                  jax.ShapeDtypeStruct((NPAD, 128), jnp.float32)],
        scratch_types=[pltpu.VMEM((8, 16), jnp.int32),
                       pltpu.VMEM((128, 128), jnp.float32),
                       pltpu.VMEM((16, D), jnp.float32),
                       pltpu.SemaphoreType.DMA,
                       pltpu.SemaphoreType.DMA],
    )
    def disp(h2_hbm, w16_hbm, pos_hbm, x_hbm, ws_hbm, posv, wbuf, rows,
             sem1, sem2):
        w = jax.lax.axis_index("s") * 2 + jax.lax.axis_index("c")
        base = w * 128
        pltpu.sync_copy(pos_hbm.at[w], posv)
        pltpu.sync_copy(w16_hbm.at[pl.ds(base, 128)], wbuf)
        for j in range(8):
            tok = jax.lax.rem(base + j * 16, T)
            pltpu.sync_copy(h2_hbm.at[pl.ds(tok, 16)], rows)
            pltpu.async_copy(rows, x_hbm.at[posv.at[j]], sem1).wait()
            pltpu.async_copy(wbuf.at[pl.ds(j * 16, 16)],
                             ws_hbm.at[posv.at[j]], sem2).wait()

    return disp(h2, w16, pos3d)


def _gmoe1_body(et_ref, x_ref, eg_ref, eu_ref, act_ref):
    g = jnp.dot(x_ref[:], eg_ref[0], preferred_element_type=jnp.float32)
    u = jnp.dot(x_ref[:], eu_ref[0], preferred_element_type=jnp.float32)
    act_ref[:] = g * jax.nn.sigmoid(g) * u


def _gmoe1(etile, x, eg, eu):
    grid_spec = pltpu.PrefetchScalarGridSpec(
        num_scalar_prefetch=1,
        grid=(NT,),
        in_specs=[
            pl.BlockSpec((BTM, D), lambda g, et: (g, 0)),
            pl.BlockSpec((1, D, F), lambda g, et: (et[g], 0, 0)),
            pl.BlockSpec((1, D, F), lambda g, et: (et[g], 0, 0)),
        ],
        out_specs=pl.BlockSpec((BTM, F), lambda g, et: (g, 0)),
    )
    return pl.pallas_call(
        _gmoe1_body,
        grid_spec=grid_spec,
        out_shape=jax.ShapeDtypeStruct((NPAD, F), jnp.float32),
        compiler_params=pltpu.CompilerParams(
            dimension_semantics=("arbitrary",)),
    )(etile, x, eg, eu)


def _gmoe2_body(et_ref, a_ref, ed_ref, ws_ref, y_ref):
    y = jnp.dot(a_ref[:], ed_ref[0], preferred_element_type=jnp.float32)
    y_ref[:] = y * ws_ref[:, :1]


def _gmoe2(etile, act, ed, ws):
    grid_spec = pltpu.PrefetchScalarGridSpec(
        num_scalar_prefetch=1,
        grid=(NT,),
        in_specs=[
            pl.BlockSpec((BTM, F), lambda g, et: (g, 0)),
            pl.BlockSpec((1, F, D), lambda g, et: (et[g], 0, 0)),
            pl.BlockSpec((BTM, 128), lambda g, et: (g, 0)),
        ],
        out_specs=pl.BlockSpec((BTM, D), lambda g, et: (g, 0)),
    )
    return pl.pallas_call(
        _gmoe2_body,
        grid_spec=grid_spec,
        out_shape=jax.ShapeDtypeStruct((NPAD, D), jnp.float32),
        compiler_params=pltpu.CompilerParams(
            dimension_semantics=("arbitrary",)),
    )(etile, act, ed, ws)


def _sc_combine(y, shared, p03d, p13d):
    from jax.experimental.pallas import tpu_sc as plsc

    mesh = plsc.VectorSubcoreMesh(core_axis_name="c", subcore_axis_name="s")

    @functools.partial(
        pl.kernel, mesh=mesh,
        out_type=jax.ShapeDtypeStruct((T, D), jnp.float32),
        scratch_types=[pltpu.VMEM((4, 16), jnp.int32),
                       pltpu.VMEM((4, 16), jnp.int32),
                       pltpu.VMEM((16, D), jnp.float32),
                       pltpu.VMEM((16, D), jnp.float32),
                       pltpu.VMEM((16, D), jnp.float32),
                       pltpu.SemaphoreType.DMA,
                       pltpu.SemaphoreType.DMA],
    )
    def comb(y_hbm, sh_hbm, p0_hbm, p1_hbm, out_hbm, p0v, p1v, y0, y1, acc,
             sem1, sem2):
        w = jax.lax.axis_index("s") * 2 + jax.lax.axis_index("c")
        pltpu.sync_copy(p0_hbm.at[w], p0v)
        pltpu.sync_copy(p1_hbm.at[w], p1v)
        for j in range(4):
            tok = w * 64 + j * 16
            cp0 = pltpu.async_copy(y_hbm.at[p0v.at[j]], y0, sem1)
            cp1 = pltpu.async_copy(y_hbm.at[p1v.at[j]], y1, sem2)
            pltpu.sync_copy(sh_hbm.at[pl.ds(tok, 16)], acc)
            cp0.wait()
            cp1.wait()
            for i in range(16):
                def body(cc, carry):
                    sl = pl.ds(cc * 16, 16)
                    acc[i, sl] = acc[i, sl] + y0[i, sl] + y1[i, sl]
                    return carry
                jax.lax.fori_loop(0, D // 16, body, 0)
            pltpu.sync_copy(acc, out_hbm.at[pl.ds(tok, 16)])

    return comb(y, shared, p03d, p13d)


def kernel(positions, hidden_states, wq, bq, wk, bk, wv, bv, wo, ln1, ln2,
           router_w, eg, eu, ed, sg, su, sd, sgw):
    half = HD // 2
    inv = 1.0 / (BASE ** (jnp.arange(half, dtype=jnp.float32) / half))
    ang = positions.astype(jnp.float32)[:, None] * inv[None, :]
    cos = jnp.cos(ang)
    sin = jnp.sin(ang)

    wqkv = jnp.concatenate([wq, wk, wv], axis=1)
    bqkv = jnp.concatenate([bq, bk, bv]).reshape(1, -1)
    ln1r = ln1.reshape(1, D)
    ln2r = ln2.reshape(1, D)

    qkv = _qkv(hidden_states, wqkv, bqkv, ln1r)
    attn = _attention(qkv, cos, sin)
    h1 = _wo_proj(attn, wo, hidden_states)
    h2, gate = _rms2(h1, ln2r, sgw)
    act_s = _shared1(h2, sg, su)
    shared = _shared2(act_s, sd, gate)

    pos0, pos1, w0, w1, etile = _router(h2, router_w)

    pos3d = jnp.concatenate([pos0, pos1], axis=0).reshape(NW, 8, 16)
    w16 = jnp.tile(jnp.concatenate([w0, w1], axis=0), (1, 128))
    p03d = pos0.reshape(NW, 4, 16)
    p13d = pos1.reshape(NW, 4, 16)

    x, ws = _sc_dispatch(h2, w16, pos3d)
    act_e = _gmoe1(etile.reshape(NT), x, eg, eu)
    y = _gmoe2(etile.reshape(NT), act_e, ed, ws)
    out = _sc_combine(y, shared, p03d, p13d)
    return (out, h1)


# speedup vs baseline: 1.2917x; 1.0115x over previous
import functools

import jax
import jax.numpy as jnp
from jax.experimental import pallas as pl
from jax.experimental.pallas import tpu as pltpu

T = 2048; D = 2048; H = 16; HD = 128; E = 8; TOPK = 2; F = 1408; FS = 5632
BASE = 1000000.0; EPS = 1e-6

BT = 256
BQ = 512
BN = 512


def _rms(x, scale):
    return x * jax.lax.rsqrt(jnp.mean(x * x, axis=-1, keepdims=True) + EPS) * scale


def _qkv_body(hs_ref, w_ref, b_ref, ln_ref, out_ref):
    h = _rms(hs_ref[:], ln_ref[:])
    out_ref[:] = jnp.dot(h, w_ref[:], preferred_element_type=jnp.float32) + b_ref[:]


def _qkv(hidden_states, wqkv, bqkv, ln1):
    grid = (T // BT, (3 * H * HD) // BN)
    return pl.pallas_call(
        _qkv_body,
        grid=grid,
        in_specs=[
            pl.BlockSpec((BT, D), lambda i, j: (i, 0)),
            pl.BlockSpec((D, BN), lambda i, j: (0, j)),
            pl.BlockSpec((1, BN), lambda i, j: (0, j)),
            pl.BlockSpec((1, D), lambda i, j: (0, 0)),
        ],
        out_specs=pl.BlockSpec((BT, BN), lambda i, j: (i, j)),
        out_shape=jax.ShapeDtypeStruct((T, 3 * H * HD), jnp.float32),
        compiler_params=pltpu.CompilerParams(
            dimension_semantics=("parallel", "parallel")),
    )(hidden_states, wqkv, bqkv, ln1)


def _rope_apply(x, cos, sin):
    x1 = x[:, :HD // 2]
    x2 = x[:, HD // 2:]
    return jnp.concatenate([x1 * cos - x2 * sin, x2 * cos + x1 * sin], axis=-1)


def _attn_body(q_ref, k_ref, v_ref, cosq_ref, sinq_ref, cos_ref, sin_ref,
               out_ref, s_ref, kr_ref, qs_ref):
    qb = pl.program_id(1)
    half = HD // 2
    scale = 1.0 / (HD ** 0.5)

    q = q_ref[:]
    q1 = q[:, :half]
    q2 = q[:, half:]
    qs_ref[:, :half] = q1 * cosq_ref[:] - q2 * sinq_ref[:]
    qs_ref[:, half:] = q2 * cosq_ref[:] + q1 * sinq_ref[:]
    k = k_ref[:]
    k1 = k[:, :half]
    k2 = k[:, half:]
    kr_ref[:, :half] = k1 * cos_ref[:] - k2 * sin_ref[:]
    kr_ref[:, half:] = k2 * cos_ref[:] + k1 * sin_ref[:]
    qr = qs_ref[:]

    row = qb * BQ + jax.lax.broadcasted_iota(jnp.int32, (BQ, BQ), 0)

    def fill(j, m):
        kj = kr_ref[pl.ds(j * BQ, BQ), :]
        s = jax.lax.dot_general(qr, kj, (((1,), (1,)), ((), ())),
                                preferred_element_type=jnp.float32) * scale
        col = j * BQ + jax.lax.broadcasted_iota(jnp.int32, (BQ, BQ), 1)
        s = jnp.where(row >= col, s, -1e30)
        s_ref[:, pl.ds(j * BQ, BQ)] = s
        return jnp.maximum(m, jnp.max(s, axis=-1, keepdims=True))

    m = jax.lax.fori_loop(0, qb + 1, fill,
                          jnp.full((BQ, 1), -1e30, jnp.float32))

    def expsum(j, l):
        p = jnp.exp(s_ref[:, pl.ds(j * BQ, BQ)] - m)
        s_ref[:, pl.ds(j * BQ, BQ)] = p
        return l + jnp.sum(p, axis=-1, keepdims=True)

    l = jax.lax.fori_loop(0, qb + 1, expsum, jnp.zeros((BQ, 1), jnp.float32))

    def accum(j, acc):
        pj = s_ref[:, pl.ds(j * BQ, BQ)] / l
        vj = v_ref[pl.ds(j * BQ, BQ), :]
        return acc + jnp.dot(pj, vj, preferred_element_type=jnp.float32)

    out_ref[:] = jax.lax.fori_loop(0, qb + 1, accum,
                                   jnp.zeros((BQ, HD), jnp.float32))


def _attention(qkv, cos, sin):
    grid = (H, T // BQ)
    return pl.pallas_call(
        _attn_body,
        grid=grid,
        in_specs=[
            pl.BlockSpec((BQ, HD), lambda h, qb: (qb, h)),
            pl.BlockSpec((T, HD), lambda h, qb: (0, H + h)),
            pl.BlockSpec((T, HD), lambda h, qb: (0, 2 * H + h)),
            pl.BlockSpec((BQ, HD // 2), lambda h, qb: (qb, 0)),
            pl.BlockSpec((BQ, HD // 2), lambda h, qb: (qb, 0)),
            pl.BlockSpec((T, HD // 2), lambda h, qb: (0, 0)),
            pl.BlockSpec((T, HD // 2), lambda h, qb: (0, 0)),
        ],
        out_specs=pl.BlockSpec((BQ, HD), lambda h, qb: (qb, h)),
        out_shape=jax.ShapeDtypeStruct((T, H * HD), jnp.float32),
        scratch_shapes=[pltpu.VMEM((BQ, T), jnp.float32),
                        pltpu.VMEM((T, HD), jnp.float32),
                        pltpu.VMEM((BQ, HD), jnp.float32)],
        compiler_params=pltpu.CompilerParams(
            dimension_semantics=("parallel", "arbitrary")),
    )(qkv, qkv, qkv, cos, sin, cos, sin)


def _wo_body(a_ref, w_ref, r_ref, out_ref):
    out_ref[:] = (jnp.dot(a_ref[:], w_ref[:], preferred_element_type=jnp.float32)
                  + r_ref[:])


def _wo_proj(attn, wo, residual):
    grid = (T // BT, D // BN)
    return pl.pallas_call(
        _wo_body,
        grid=grid,
        in_specs=[
            pl.BlockSpec((BT, H * HD), lambda i, j: (i, 0)),
            pl.BlockSpec((H * HD, BN), lambda i, j: (0, j)),
            pl.BlockSpec((BT, BN), lambda i, j: (i, j)),
        ],
        out_specs=pl.BlockSpec((BT, BN), lambda i, j: (i, j)),
        out_shape=jax.ShapeDtypeStruct((T, D), jnp.float32),
        compiler_params=pltpu.CompilerParams(
            dimension_semantics=("parallel", "parallel")),
    )(attn, wo, residual)


def _rms2_body(h_ref, ln_ref, sgw_ref, h2_ref, gate_ref):
    h2 = _rms(h_ref[:], ln_ref[:])
    h2_ref[:] = h2
    gate_ref[:] = jax.nn.sigmoid(
        jnp.dot(h2, sgw_ref[:], preferred_element_type=jnp.float32))


def _rms2(h, ln2, sgw):
    grid = (T // BT,)
    return pl.pallas_call(
        _rms2_body,
        grid=grid,
        in_specs=[
            pl.BlockSpec((BT, D), lambda i: (i, 0)),
            pl.BlockSpec((1, D), lambda i: (0, 0)),
            pl.BlockSpec((D, 1), lambda i: (0, 0)),
        ],
        out_specs=[
            pl.BlockSpec((BT, D), lambda i: (i, 0)),
            pl.BlockSpec((BT, 1), lambda i: (i, 0)),
        ],
        out_shape=[
            jax.ShapeDtypeStruct((T, D), jnp.float32),
            jax.ShapeDtypeStruct((T, 1), jnp.float32),
        ],
        compiler_params=pltpu.CompilerParams(
            dimension_semantics=("parallel",)),
    )(h, ln2, sgw)


def _sh1_body(h2_ref, sg_ref, su_ref, act_ref):
    g = jnp.dot(h2_ref[:], sg_ref[:], preferred_element_type=jnp.float32)
    u = jnp.dot(h2_ref[:], su_ref[:], preferred_element_type=jnp.float32)
    act_ref[:] = g * jax.nn.sigmoid(g) * u


def _shared1(h2, sg, su):
    grid = (FS // BN, T // BQ)
    return pl.pallas_call(
        _sh1_body,
        grid=grid,
        in_specs=[
            pl.BlockSpec((BQ, D), lambda j, i: (i, 0)),
            pl.BlockSpec((D, BN), lambda j, i: (0, j)),
            pl.BlockSpec((D, BN), lambda j, i: (0, j)),
        ],
        out_specs=pl.BlockSpec((BQ, BN), lambda j, i: (i, j)),
        out_shape=jax.ShapeDtypeStruct((T, FS), jnp.float32),
        compiler_params=pltpu.CompilerParams(
            dimension_semantics=("arbitrary", "arbitrary")),
    )(h2, sg, su)


def _sh2_body(a_ref, sd_ref, gate_ref, out_ref):
    out_ref[:] = gate_ref[:] * jnp.dot(a_ref[:], sd_ref[:],
                                       preferred_element_type=jnp.float32)


def _shared2(act, sd, gate):
    grid = (D // BN, T // BT)
    return pl.pallas_call(
        _sh2_body,
        grid=grid,
        in_specs=[
            pl.BlockSpec((BT, FS), lambda j, i: (i, 0)),
            pl.BlockSpec((FS, BN), lambda j, i: (0, j)),
            pl.BlockSpec((BT, 1), lambda j, i: (i, 0)),
        ],
        out_specs=pl.BlockSpec((BT, BN), lambda j, i: (i, j)),
        out_shape=jax.ShapeDtypeStruct((T, D), jnp.float32),
        compiler_params=pltpu.CompilerParams(
            dimension_semantics=("arbitrary", "arbitrary")),
    )(act, sd, gate)


BTM = 256
NT = 2 * T // BTM + E
NPAD = NT * BTM
NW = 32


def _cumsum_rows(x):
    n = x.shape[0]
    sh = 1
    while sh < n:
        x = x + jnp.concatenate(
            [jnp.zeros((sh, x.shape[1]), x.dtype), x[:-sh]], axis=0)
        sh *= 2
    return x


def _router_body(h2_ref, rw_ref, pos0_ref, pos1_ref, w0_ref, w1_ref, et_ref):
    logits = jnp.dot(h2_ref[:], rw_ref[:], preferred_element_type=jnp.float32)
    p = jax.nn.softmax(logits, axis=-1)
    iota = jax.lax.broadcasted_iota(jnp.int32, p.shape, 1)
    m1 = jnp.max(p, axis=-1, keepdims=True)
    i1 = jnp.min(jnp.where(p == m1, iota, E), axis=-1, keepdims=True)
    p2 = jnp.where(iota == i1, -1.0, p)
    m2 = jnp.max(p2, axis=-1, keepdims=True)
    i2 = jnp.min(jnp.where(p2 == m2, iota, E), axis=-1, keepdims=True)
    denom = m1 + m2
    w0_ref[:] = m1 / denom
    w1_ref[:] = m2 / denom

    oh0 = (iota == i1).astype(jnp.float32)
    oh1 = (iota == i2).astype(jnp.float32)
    inc0 = _cumsum_rows(oh0)
    inc1 = _cumsum_rows(oh1)
    cnt0 = inc0[T - 1:, :]
    counts = cnt0 + inc1[T - 1:, :]
    counts_i = counts.astype(jnp.int32)
    tiles = ((counts_i + (BTM - 1)) // BTM).astype(jnp.float32)

    r = jax.lax.broadcasted_iota(jnp.int32, (E, E), 0)
    c = jax.lax.broadcasted_iota(jnp.int32, (E, E), 1)
    ustrict = (r < c).astype(jnp.float32)
    uincl = (r <= c).astype(jnp.float32)
    start_tile = jnp.dot(tiles, ustrict, preferred_element_type=jnp.float32)
    end_tile = jnp.dot(tiles, uincl, preferred_element_type=jnp.float32)
    pad_start = start_tile * float(BTM)

    rank0 = inc0 - oh0
    rank1 = cnt0 + inc1 - oh1
    pos0 = jnp.sum(oh0 * (pad_start + rank0), axis=1, keepdims=True)
    pos1 = jnp.sum(oh1 * (pad_start + rank1), axis=1, keepdims=True)
    pos0_ref[:] = pos0.astype(jnp.int32)
    pos1_ref[:] = pos1.astype(jnp.int32)

    g = jax.lax.broadcasted_iota(jnp.int32, (NT, E), 0)
    ind = (g >= end_tile.astype(jnp.int32)).astype(jnp.float32)
    et = jnp.sum(ind, axis=1, keepdims=True).astype(jnp.int32)
    et_ref[:] = jnp.minimum(et, E - 1)


def _router(h2, router_w):
    return pl.pallas_call(
        _router_body,
        grid=(1,),
        in_specs=[
            pl.BlockSpec((T, D), lambda i: (0, 0)),
            pl.BlockSpec((D, E), lambda i: (0, 0)),
        ],
        out_specs=[
            pl.BlockSpec((T, 1), lambda i: (0, 0)),
            pl.BlockSpec((T, 1), lambda i: (0, 0)),
            pl.BlockSpec((T, 1), lambda i: (0, 0)),
            pl.BlockSpec((T, 1), lambda i: (0, 0)),
            pl.BlockSpec((NT, 1), lambda i: (0, 0)),
        ],
        out_shape=[
            jax.ShapeDtypeStruct((T, 1), jnp.int32),
            jax.ShapeDtypeStruct((T, 1), jnp.int32),
            jax.ShapeDtypeStruct((T, 1), jnp.float32),
            jax.ShapeDtypeStruct((T, 1), jnp.float32),
            jax.ShapeDtypeStruct((NT, 1), jnp.int32),
        ],
        compiler_params=pltpu.CompilerParams(
            dimension_semantics=("arbitrary",)),
    )(h2, router_w)


def _sc_dispatch(h2, w16, pos3d):
    from jax.experimental.pallas import tpu_sc as plsc

    mesh = plsc.VectorSubcoreMesh(core_axis_name="c", subcore_axis_name="s")

    @functools.partial(
        pl.kernel, mesh=mesh,
        out_type=[jax.ShapeDtypeStruct((NPAD, D), jnp.float32),
                  jax.ShapeDtypeStruct((NPAD, 128), jnp.float32)],
        scratch_types=[pltpu.VMEM((8, 16), jnp.int32),
                       pltpu.VMEM((128, 128), jnp.float32),
                       pltpu.VMEM((16, D), jnp.float32),
                       pltpu.SemaphoreType.DMA,
                       pltpu.SemaphoreType.DMA],
    )
    def disp(h2_hbm, w16_hbm, pos_hbm, x_hbm, ws_hbm, posv, wbuf, rows,
             sem1, sem2):
        w = jax.lax.axis_index("s") * 2 + jax.lax.axis_index("c")
        base = w * 128
        pltpu.sync_copy(pos_hbm.at[w], posv)
        pltpu.sync_copy(w16_hbm.at[pl.ds(base, 128)], wbuf)
        for j in range(8):
            tok = jax.lax.rem(base + j * 16, T)
            pltpu.sync_copy(h2_hbm.at[pl.ds(tok, 16)], rows)
            pltpu.async_copy(rows, x_hbm.at[posv.at[j]], sem1).wait()
            pltpu.async_copy(wbuf.at[pl.ds(j * 16, 16)],
                             ws_hbm.at[posv.at[j]], sem2).wait()

    return disp(h2, w16, pos3d)


def _gmoe1_body(et_ref, x_ref, eg_ref, eu_ref, act_ref):
    g = jnp.dot(x_ref[:], eg_ref[0], preferred_element_type=jnp.float32)
    u = jnp.dot(x_ref[:], eu_ref[0], preferred_element_type=jnp.float32)
    act_ref[:] = g * jax.nn.sigmoid(g) * u


def _gmoe1(etile, x, eg, eu):
    grid_spec = pltpu.PrefetchScalarGridSpec(
        num_scalar_prefetch=1,
        grid=(NT,),
        in_specs=[
            pl.BlockSpec((BTM, D), lambda g, et: (g, 0)),
            pl.BlockSpec((1, D, F), lambda g, et: (et[g], 0, 0)),
            pl.BlockSpec((1, D, F), lambda g, et: (et[g], 0, 0)),
        ],
        out_specs=pl.BlockSpec((BTM, F), lambda g, et: (g, 0)),
    )
    return pl.pallas_call(
        _gmoe1_body,
        grid_spec=grid_spec,
        out_shape=jax.ShapeDtypeStruct((NPAD, F), jnp.float32),
        compiler_params=pltpu.CompilerParams(
            dimension_semantics=("arbitrary",)),
    )(etile, x, eg, eu)


def _gmoe2_body(et_ref, a_ref, ed_ref, ws_ref, y_ref):
    y = jnp.dot(a_ref[:], ed_ref[0], preferred_element_type=jnp.float32)
    y_ref[:] = y * ws_ref[:, :1]


def _gmoe2(etile, act, ed, ws):
    grid_spec = pltpu.PrefetchScalarGridSpec(
        num_scalar_prefetch=1,
        grid=(NT,),
        in_specs=[
            pl.BlockSpec((BTM, F), lambda g, et: (g, 0)),
            pl.BlockSpec((1, F, D), lambda g, et: (et[g], 0, 0)),
            pl.BlockSpec((BTM, 128), lambda g, et: (g, 0)),
        ],
        out_specs=pl.BlockSpec((BTM, D), lambda g, et: (g, 0)),
    )
    return pl.pallas_call(
        _gmoe2_body,
        grid_spec=grid_spec,
        out_shape=jax.ShapeDtypeStruct((NPAD, D), jnp.float32),
        compiler_params=pltpu.CompilerParams(
            dimension_semantics=("arbitrary",)),
    )(etile, act, ed, ws)


def _sc_combine(y, shared, p03d, p13d):
    from jax.experimental.pallas import tpu_sc as plsc

    mesh = plsc.VectorSubcoreMesh(core_axis_name="c", subcore_axis_name="s")

    @functools.partial(
        pl.kernel, mesh=mesh,
        out_type=jax.ShapeDtypeStruct((T, D), jnp.float32),
        scratch_types=[pltpu.VMEM((4, 16), jnp.int32),
                       pltpu.VMEM((4, 16), jnp.int32),
                       pltpu.VMEM((16, D), jnp.float32),
                       pltpu.VMEM((16, D), jnp.float32),
                       pltpu.VMEM((16, D), jnp.float32),
                       pltpu.SemaphoreType.DMA,
                       pltpu.SemaphoreType.DMA],
    )
    def comb(y_hbm, sh_hbm, p0_hbm, p1_hbm, out_hbm, p0v, p1v, y0, y1, acc,
             sem1, sem2):
        w = jax.lax.axis_index("s") * 2 + jax.lax.axis_index("c")
        pltpu.sync_copy(p0_hbm.at[w], p0v)
        pltpu.sync_copy(p1_hbm.at[w], p1v)
        for j in range(4):
            tok = w * 64 + j * 16
            cp0 = pltpu.async_copy(y_hbm.at[p0v.at[j]], y0, sem1)
            cp1 = pltpu.async_copy(y_hbm.at[p1v.at[j]], y1, sem2)
            pltpu.sync_copy(sh_hbm.at[pl.ds(tok, 16)], acc)
            cp0.wait()
            cp1.wait()
            for i in range(16):
                def body(cc, carry):
                    sl = pl.ds(cc * 16, 16)
                    acc[i, sl] = acc[i, sl] + y0[i, sl] + y1[i, sl]
                    return carry
                jax.lax.fori_loop(0, D // 16, body, 0)
            pltpu.sync_copy(acc, out_hbm.at[pl.ds(tok, 16)])

    return comb(y, shared, p03d, p13d)


def kernel(positions, hidden_states, wq, bq, wk, bk, wv, bv, wo, ln1, ln2,
           router_w, eg, eu, ed, sg, su, sd, sgw):
    half = HD // 2
    inv = 1.0 / (BASE ** (jnp.arange(half, dtype=jnp.float32) / half))
    ang = positions.astype(jnp.float32)[:, None] * inv[None, :]
    cos = jnp.cos(ang)
    sin = jnp.sin(ang)

    wqkv = jnp.concatenate([wq, wk, wv], axis=1)
    bqkv = jnp.concatenate([bq, bk, bv]).reshape(1, -1)
    ln1r = ln1.reshape(1, D)
    ln2r = ln2.reshape(1, D)

    qkv = _qkv(hidden_states, wqkv, bqkv, ln1r)
    attn = _attention(qkv, cos, sin)
    h1 = _wo_proj(attn, wo, hidden_states)
    h2, gate = _rms2(h1, ln2r, sgw)
    act_s = _shared1(h2, sg, su)
    shared = _shared2(act_s, sd, gate)

    pos0, pos1, w0, w1, etile = _router(h2, router_w)

    pos3d = jnp.concatenate([pos0, pos1], axis=0).reshape(NW, 8, 16)
    w16 = jnp.tile(jnp.concatenate([w0, w1], axis=0), (1, 128))
    p03d = pos0.reshape(NW, 4, 16)
    p13d = pos1.reshape(NW, 4, 16)

    x, ws = _sc_dispatch(h2, w16, pos3d)
    act_e = _gmoe1(etile.reshape(NT), x, eg, eu)
    y = _gmoe2(etile.reshape(NT), act_e, ed, ws)
    out = _sc_combine(y, shared, p03d, p13d)
    return (out, h1)


# speedup vs baseline: 1.3272x; 1.0275x over previous
import functools

import jax
import jax.numpy as jnp
from jax.experimental import pallas as pl
from jax.experimental.pallas import tpu as pltpu

T = 2048; D = 2048; H = 16; HD = 128; E = 8; TOPK = 2; F = 1408; FS = 5632
BASE = 1000000.0; EPS = 1e-6

BT = 256
BQ = 512
BN = 512


def _rms(x, scale):
    return x * jax.lax.rsqrt(jnp.mean(x * x, axis=-1, keepdims=True) + EPS) * scale


def _qkv_body(hs_ref, w_ref, b_ref, ln_ref, out_ref):
    h = _rms(hs_ref[:], ln_ref[:])
    out_ref[:] = jnp.dot(h, w_ref[:], preferred_element_type=jnp.float32) + b_ref[:]


def _qkv(hidden_states, wqkv, bqkv, ln1):
    grid = (T // BT, (3 * H * HD) // BN)
    return pl.pallas_call(
        _qkv_body,
        grid=grid,
        in_specs=[
            pl.BlockSpec((BT, D), lambda i, j: (i, 0)),
            pl.BlockSpec((D, BN), lambda i, j: (0, j)),
            pl.BlockSpec((1, BN), lambda i, j: (0, j)),
            pl.BlockSpec((1, D), lambda i, j: (0, 0)),
        ],
        out_specs=pl.BlockSpec((BT, BN), lambda i, j: (i, j)),
        out_shape=jax.ShapeDtypeStruct((T, 3 * H * HD), jnp.float32),
        compiler_params=pltpu.CompilerParams(
            dimension_semantics=("parallel", "parallel")),
    )(hidden_states, wqkv, bqkv, ln1)


def _rope_apply(x, cos, sin):
    x1 = x[:, :HD // 2]
    x2 = x[:, HD // 2:]
    return jnp.concatenate([x1 * cos - x2 * sin, x2 * cos + x1 * sin], axis=-1)


def _attn_body(q_ref, k_ref, v_ref, cosq_ref, sinq_ref, cos_ref, sin_ref,
               out_ref, s_ref, kr_ref, qs_ref):
    qb = pl.program_id(1)
    half = HD // 2
    scale = 1.0 / (HD ** 0.5)

    q = q_ref[:]
    q1 = q[:, :half]
    q2 = q[:, half:]
    qs_ref[:, :half] = q1 * cosq_ref[:] - q2 * sinq_ref[:]
    qs_ref[:, half:] = q2 * cosq_ref[:] + q1 * sinq_ref[:]
    k = k_ref[:]
    k1 = k[:, :half]
    k2 = k[:, half:]
    kr_ref[:, :half] = k1 * cos_ref[:] - k2 * sin_ref[:]
    kr_ref[:, half:] = k2 * cos_ref[:] + k1 * sin_ref[:]
    qr = qs_ref[:]

    row = qb * BQ + jax.lax.broadcasted_iota(jnp.int32, (BQ, BQ), 0)

    def fill(j, m):
        kj = kr_ref[pl.ds(j * BQ, BQ), :]
        s = jax.lax.dot_general(qr, kj, (((1,), (1,)), ((), ())),
                                preferred_element_type=jnp.float32) * scale
        col = j * BQ + jax.lax.broadcasted_iota(jnp.int32, (BQ, BQ), 1)
        s = jnp.where(row >= col, s, -1e30)
        s_ref[:, pl.ds(j * BQ, BQ)] = s
        return jnp.maximum(m, jnp.max(s, axis=-1, keepdims=True))

    m = jax.lax.fori_loop(0, qb + 1, fill,
                          jnp.full((BQ, 1), -1e30, jnp.float32))

    def expsum(j, l):
        p = jnp.exp(s_ref[:, pl.ds(j * BQ, BQ)] - m)
        s_ref[:, pl.ds(j * BQ, BQ)] = p
        return l + jnp.sum(p, axis=-1, keepdims=True)

    l = jax.lax.fori_loop(0, qb + 1, expsum, jnp.zeros((BQ, 1), jnp.float32))

    def accum(j, acc):
        pj = s_ref[:, pl.ds(j * BQ, BQ)] / l
        vj = v_ref[pl.ds(j * BQ, BQ), :]
        return acc + jnp.dot(pj, vj, preferred_element_type=jnp.float32)

    out_ref[:] = jax.lax.fori_loop(0, qb + 1, accum,
                                   jnp.zeros((BQ, HD), jnp.float32))


def _attention(qkv, cos, sin):
    grid = (H, T // BQ)
    return pl.pallas_call(
        _attn_body,
        grid=grid,
        in_specs=[
            pl.BlockSpec((BQ, HD), lambda h, qb: (qb, h)),
            pl.BlockSpec((T, HD), lambda h, qb: (0, H + h)),
            pl.BlockSpec((T, HD), lambda h, qb: (0, 2 * H + h)),
            pl.BlockSpec((BQ, HD // 2), lambda h, qb: (qb, 0)),
            pl.BlockSpec((BQ, HD // 2), lambda h, qb: (qb, 0)),
            pl.BlockSpec((T, HD // 2), lambda h, qb: (0, 0)),
            pl.BlockSpec((T, HD // 2), lambda h, qb: (0, 0)),
        ],
        out_specs=pl.BlockSpec((BQ, HD), lambda h, qb: (qb, h)),
        out_shape=jax.ShapeDtypeStruct((T, H * HD), jnp.float32),
        scratch_shapes=[pltpu.VMEM((BQ, T), jnp.float32),
                        pltpu.VMEM((T, HD), jnp.float32),
                        pltpu.VMEM((BQ, HD), jnp.float32)],
        compiler_params=pltpu.CompilerParams(
            dimension_semantics=("parallel", "arbitrary")),
    )(qkv, qkv, qkv, cos, sin, cos, sin)


def _wo_body(a_ref, w_ref, r_ref, out_ref):
    out_ref[:] = (jnp.dot(a_ref[:], w_ref[:], preferred_element_type=jnp.float32)
                  + r_ref[:])


def _wo_proj(attn, wo, residual):
    grid = (T // BT, D // BN)
    return pl.pallas_call(
        _wo_body,
        grid=grid,
        in_specs=[
            pl.BlockSpec((BT, H * HD), lambda i, j: (i, 0)),
            pl.BlockSpec((H * HD, BN), lambda i, j: (0, j)),
            pl.BlockSpec((BT, BN), lambda i, j: (i, j)),
        ],
        out_specs=pl.BlockSpec((BT, BN), lambda i, j: (i, j)),
        out_shape=jax.ShapeDtypeStruct((T, D), jnp.float32),
        compiler_params=pltpu.CompilerParams(
            dimension_semantics=("parallel", "parallel")),
    )(attn, wo, residual)


def _rms2_body(h_ref, ln_ref, sgw_ref, h2_ref, gate_ref):
    h2 = _rms(h_ref[:], ln_ref[:])
    h2_ref[:] = h2
    gate_ref[:] = jax.nn.sigmoid(
        jnp.dot(h2, sgw_ref[:], preferred_element_type=jnp.float32))


def _rms2(h, ln2, sgw):
    grid = (T // BT,)
    return pl.pallas_call(
        _rms2_body,
        grid=grid,
        in_specs=[
            pl.BlockSpec((BT, D), lambda i: (i, 0)),
            pl.BlockSpec((1, D), lambda i: (0, 0)),
            pl.BlockSpec((D, 1), lambda i: (0, 0)),
        ],
        out_specs=[
            pl.BlockSpec((BT, D), lambda i: (i, 0)),
            pl.BlockSpec((BT, 1), lambda i: (i, 0)),
        ],
        out_shape=[
            jax.ShapeDtypeStruct((T, D), jnp.float32),
            jax.ShapeDtypeStruct((T, 1), jnp.float32),
        ],
        compiler_params=pltpu.CompilerParams(
            dimension_semantics=("parallel",)),
    )(h, ln2, sgw)


def _sh1_body(h2_ref, sg_ref, su_ref, act_ref):
    g = jnp.dot(h2_ref[:], sg_ref[:], preferred_element_type=jnp.float32)
    u = jnp.dot(h2_ref[:], su_ref[:], preferred_element_type=jnp.float32)
    act_ref[:] = g * jax.nn.sigmoid(g) * u


def _shared1(h2, sg, su):
    grid = (FS // BN, T // BQ)
    return pl.pallas_call(
        _sh1_body,
        grid=grid,
        in_specs=[
            pl.BlockSpec((BQ, D), lambda j, i: (i, 0)),
            pl.BlockSpec((D, BN), lambda j, i: (0, j)),
            pl.BlockSpec((D, BN), lambda j, i: (0, j)),
        ],
        out_specs=pl.BlockSpec((BQ, BN), lambda j, i: (i, j)),
        out_shape=jax.ShapeDtypeStruct((T, FS), jnp.float32),
        compiler_params=pltpu.CompilerParams(
            dimension_semantics=("arbitrary", "arbitrary")),
    )(h2, sg, su)


def _sh2_body(a_ref, sd_ref, gate_ref, out_ref):
    out_ref[:] = gate_ref[:] * jnp.dot(a_ref[:], sd_ref[:],
                                       preferred_element_type=jnp.float32)


def _shared2(act, sd, gate):
    grid = (D // BN, T // BT)
    return pl.pallas_call(
        _sh2_body,
        grid=grid,
        in_specs=[
            pl.BlockSpec((BT, FS), lambda j, i: (i, 0)),
            pl.BlockSpec((FS, BN), lambda j, i: (0, j)),
            pl.BlockSpec((BT, 1), lambda j, i: (i, 0)),
        ],
        out_specs=pl.BlockSpec((BT, BN), lambda j, i: (i, j)),
        out_shape=jax.ShapeDtypeStruct((T, D), jnp.float32),
        compiler_params=pltpu.CompilerParams(
            dimension_semantics=("arbitrary", "arbitrary")),
    )(act, sd, gate)


BTM = 256
NT = 2 * T // BTM + E
NPAD = NT * BTM
NW = 32


def _cumsum_rows(x):
    n = x.shape[0]
    sh = 1
    while sh < n:
        x = x + jnp.concatenate(
            [jnp.zeros((sh, x.shape[1]), x.dtype), x[:-sh]], axis=0)
        sh *= 2
    return x


def _router_body(h2_ref, rw_ref, pos0_ref, pos1_ref, w0_ref, w1_ref, et_ref):
    logits = jnp.dot(h2_ref[:], rw_ref[:], preferred_element_type=jnp.float32)
    p = jax.nn.softmax(logits, axis=-1)
    iota = jax.lax.broadcasted_iota(jnp.int32, p.shape, 1)
    m1 = jnp.max(p, axis=-1, keepdims=True)
    i1 = jnp.min(jnp.where(p == m1, iota, E), axis=-1, keepdims=True)
    p2 = jnp.where(iota == i1, -1.0, p)
    m2 = jnp.max(p2, axis=-1, keepdims=True)
    i2 = jnp.min(jnp.where(p2 == m2, iota, E), axis=-1, keepdims=True)
    denom = m1 + m2
    w0_ref[:] = m1 / denom
    w1_ref[:] = m2 / denom

    oh0 = (iota == i1).astype(jnp.float32)
    oh1 = (iota == i2).astype(jnp.float32)
    inc0 = _cumsum_rows(oh0)
    inc1 = _cumsum_rows(oh1)
    cnt0 = inc0[T - 1:, :]
    counts = cnt0 + inc1[T - 1:, :]
    counts_i = counts.astype(jnp.int32)
    tiles = ((counts_i + (BTM - 1)) // BTM).astype(jnp.float32)

    r = jax.lax.broadcasted_iota(jnp.int32, (E, E), 0)
    c = jax.lax.broadcasted_iota(jnp.int32, (E, E), 1)
    ustrict = (r < c).astype(jnp.float32)
    uincl = (r <= c).astype(jnp.float32)
    start_tile = jnp.dot(tiles, ustrict, preferred_element_type=jnp.float32)
    end_tile = jnp.dot(tiles, uincl, preferred_element_type=jnp.float32)
    pad_start = start_tile * float(BTM)

    rank0 = inc0 - oh0
    rank1 = cnt0 + inc1 - oh1
    pos0 = jnp.sum(oh0 * (pad_start + rank0), axis=1, keepdims=True)
    pos1 = jnp.sum(oh1 * (pad_start + rank1), axis=1, keepdims=True)
    pos0_ref[:] = pos0.astype(jnp.int32)
    pos1_ref[:] = pos1.astype(jnp.int32)

    g = jax.lax.broadcasted_iota(jnp.int32, (NT, E), 0)
    ind = (g >= end_tile.astype(jnp.int32)).astype(jnp.float32)
    et = jnp.sum(ind, axis=1, keepdims=True).astype(jnp.int32)
    et_ref[:] = jnp.minimum(et, E - 1)


def _router(h2, router_w):
    return pl.pallas_call(
        _router_body,
        grid=(1,),
        in_specs=[
            pl.BlockSpec((T, D), lambda i: (0, 0)),
            pl.BlockSpec((D, E), lambda i: (0, 0)),
        ],
        out_specs=[
            pl.BlockSpec((T, 1), lambda i: (0, 0)),
            pl.BlockSpec((T, 1), lambda i: (0, 0)),
            pl.BlockSpec((T, 1), lambda i: (0, 0)),
            pl.BlockSpec((T, 1), lambda i: (0, 0)),
            pl.BlockSpec((NT, 1), lambda i: (0, 0)),
        ],
        out_shape=[
            jax.ShapeDtypeStruct((T, 1), jnp.int32),
            jax.ShapeDtypeStruct((T, 1), jnp.int32),
            jax.ShapeDtypeStruct((T, 1), jnp.float32),
            jax.ShapeDtypeStruct((T, 1), jnp.float32),
            jax.ShapeDtypeStruct((NT, 1), jnp.int32),
        ],
        compiler_params=pltpu.CompilerParams(
            dimension_semantics=("arbitrary",)),
    )(h2, router_w)


def _sc_dispatch(h2, w16, pos3d):
    from jax.experimental.pallas import tpu_sc as plsc

    mesh = plsc.VectorSubcoreMesh(core_axis_name="c", subcore_axis_name="s")

    @functools.partial(
        pl.kernel, mesh=mesh,
        out_type=[jax.ShapeDtypeStruct((NPAD, D), jnp.float32),
                  jax.ShapeDtypeStruct((NPAD, 128), jnp.float32)],
        scratch_types=[pltpu.VMEM((8, 16), jnp.int32),
                       pltpu.VMEM((128, 128), jnp.float32),
                       pltpu.VMEM((16, D), jnp.float32),
                       pltpu.SemaphoreType.DMA,
                       pltpu.SemaphoreType.DMA],
    )
    def disp(h2_hbm, w16_hbm, pos_hbm, x_hbm, ws_hbm, posv, wbuf, rows,
             sem1, sem2):
        w = jax.lax.axis_index("s") * 2 + jax.lax.axis_index("c")
        base = w * 128
        pltpu.sync_copy(pos_hbm.at[w], posv)
        pltpu.sync_copy(w16_hbm.at[pl.ds(base, 128)], wbuf)
        for j in range(8):
            tok = jax.lax.rem(base + j * 16, T)
            pltpu.sync_copy(h2_hbm.at[pl.ds(tok, 16)], rows)
            pltpu.async_copy(rows, x_hbm.at[posv.at[j]], sem1).wait()
            pltpu.async_copy(wbuf.at[pl.ds(j * 16, 16)],
                             ws_hbm.at[posv.at[j]], sem2).wait()

    return disp(h2, w16, pos3d)


def _gmoe1_body(et_ref, x_ref, eg_ref, eu_ref, act_ref):
    g = jnp.dot(x_ref[:], eg_ref[0], preferred_element_type=jnp.float32)
    u = jnp.dot(x_ref[:], eu_ref[0], preferred_element_type=jnp.float32)
    act_ref[:] = g * jax.nn.sigmoid(g) * u


def _gmoe1(etile, x, eg, eu):
    grid_spec = pltpu.PrefetchScalarGridSpec(
        num_scalar_prefetch=1,
        grid=(NT,),
        in_specs=[
            pl.BlockSpec((BTM, D), lambda g, et: (g, 0)),
            pl.BlockSpec((1, D, F), lambda g, et: (et[g], 0, 0)),
            pl.BlockSpec((1, D, F), lambda g, et: (et[g], 0, 0)),
        ],
        out_specs=pl.BlockSpec((BTM, F), lambda g, et: (g, 0)),
    )
    return pl.pallas_call(
        _gmoe1_body,
        grid_spec=grid_spec,
        out_shape=jax.ShapeDtypeStruct((NPAD, F), jnp.float32),
        compiler_params=pltpu.CompilerParams(
            dimension_semantics=("arbitrary",)),
    )(etile, x, eg, eu)


def _gmoe2_body(et_ref, a_ref, ed_ref, ws_ref, y_ref):
    y = jnp.dot(a_ref[:], ed_ref[0], preferred_element_type=jnp.float32)
    y_ref[:] = y * ws_ref[:, :1]


def _gmoe2(etile, act, ed, ws):
    grid_spec = pltpu.PrefetchScalarGridSpec(
        num_scalar_prefetch=1,
        grid=(NT,),
        in_specs=[
            pl.BlockSpec((BTM, F), lambda g, et: (g, 0)),
            pl.BlockSpec((1, F, D), lambda g, et: (et[g], 0, 0)),
            pl.BlockSpec((BTM, 128), lambda g, et: (g, 0)),
        ],
        out_specs=pl.BlockSpec((BTM, D), lambda g, et: (g, 0)),
    )
    return pl.pallas_call(
        _gmoe2_body,
        grid_spec=grid_spec,
        out_shape=jax.ShapeDtypeStruct((NPAD, D), jnp.float32),
        compiler_params=pltpu.CompilerParams(
            dimension_semantics=("arbitrary",)),
    )(etile, act, ed, ws)


def _sc_combine(y, shared, p03d, p13d):
    from jax.experimental.pallas import tpu_sc as plsc

    mesh = plsc.VectorSubcoreMesh(core_axis_name="c", subcore_axis_name="s")

    @functools.partial(
        pl.kernel, mesh=mesh,
        out_type=jax.ShapeDtypeStruct((T, D), jnp.float32),
        scratch_types=[pltpu.VMEM((4, 16), jnp.int32),
                       pltpu.VMEM((4, 16), jnp.int32),
                       pltpu.VMEM((16, D), jnp.float32),
                       pltpu.VMEM((16, D), jnp.float32),
                       pltpu.VMEM((16, D), jnp.float32),
                       pltpu.SemaphoreType.DMA,
                       pltpu.SemaphoreType.DMA],
    )
    def comb(y_hbm, sh_hbm, p0_hbm, p1_hbm, out_hbm, p0v, p1v, y0, y1, acc,
             sem1, sem2):
        w = jax.lax.axis_index("s") * 2 + jax.lax.axis_index("c")
        pltpu.sync_copy(p0_hbm.at[w], p0v)
        pltpu.sync_copy(p1_hbm.at[w], p1v)
        for j in range(4):
            tok = w * 64 + j * 16
            cp0 = pltpu.async_copy(y_hbm.at[p0v.at[j]], y0, sem1)
            cp1 = pltpu.async_copy(y_hbm.at[p1v.at[j]], y1, sem2)
            pltpu.sync_copy(sh_hbm.at[pl.ds(tok, 16)], acc)
            cp0.wait()
            cp1.wait()
            for i in range(16):
                def body(cc, carry):
                    for u in range(8):
                        sl = pl.ds(cc * 128 + u * 16, 16)
                        acc[i, sl] = acc[i, sl] + y0[i, sl] + y1[i, sl]
                    return carry
                jax.lax.fori_loop(0, D // 128, body, 0)
            pltpu.sync_copy(acc, out_hbm.at[pl.ds(tok, 16)])

    return comb(y, shared, p03d, p13d)


def kernel(positions, hidden_states, wq, bq, wk, bk, wv, bv, wo, ln1, ln2,
           router_w, eg, eu, ed, sg, su, sd, sgw):
    half = HD // 2
    inv = 1.0 / (BASE ** (jnp.arange(half, dtype=jnp.float32) / half))
    ang = positions.astype(jnp.float32)[:, None] * inv[None, :]
    cos = jnp.cos(ang)
    sin = jnp.sin(ang)

    wqkv = jnp.concatenate([wq, wk, wv], axis=1)
    bqkv = jnp.concatenate([bq, bk, bv]).reshape(1, -1)
    ln1r = ln1.reshape(1, D)
    ln2r = ln2.reshape(1, D)

    qkv = _qkv(hidden_states, wqkv, bqkv, ln1r)
    attn = _attention(qkv, cos, sin)
    h1 = _wo_proj(attn, wo, hidden_states)
    h2, gate = _rms2(h1, ln2r, sgw)
    act_s = _shared1(h2, sg, su)
    shared = _shared2(act_s, sd, gate)

    pos0, pos1, w0, w1, etile = _router(h2, router_w)

    pos3d = jnp.concatenate([pos0, pos1], axis=0).reshape(NW, 8, 16)
    w16 = jnp.tile(jnp.concatenate([w0, w1], axis=0), (1, 128))
    p03d = pos0.reshape(NW, 4, 16)
    p13d = pos1.reshape(NW, 4, 16)

    x, ws = _sc_dispatch(h2, w16, pos3d)
    act_e = _gmoe1(etile.reshape(NT), x, eg, eu)
    y = _gmoe2(etile.reshape(NT), act_e, ed, ws)
    out = _sc_combine(y, shared, p03d, p13d)
    return (out, h1)
